# baseline (speedup 1.0000x reference)
import functools

import jax
import jax.numpy as jnp
import numpy as np
from jax import lax
from jax.experimental import pallas as pl
from jax.experimental.pallas import tpu as pltpu
from jax.experimental.pallas import tpu_sc as plsc

F32 = jnp.float32
_N = 10000
_E = 160000
_E2 = 2 * _E
_EMB = 32
_R_MAX = 10.0
_NB = 1000
_EB = 1000
_LMAP = [0, 1, 1, 1, 2, 2, 2, 2, 2]
_S3 = 3.0 ** 0.5
_S15 = 15.0 ** 0.5
_S5 = 5.0 ** 0.5

_NC, _NS = 2, 16
_NW = _NC * _NS
_GPER = _E // _NW
_SPER = _E2 // _NS
_SCH = 128
_GFULL = _GPER // _SCH
_GTAIL = _GPER - _GFULL * _SCH
_MCH = 64
_STAIL = 32
_SFULL = (_SPER - _STAIL) // _MCH
_PPER = _E2 // _NW
_PFULL = _PPER // _SCH
_PTAIL = _PPER - _PFULL * _SCH
_NPAD = 10112
_NROWS = _NPAD // _NS


def _mesh():
    return plsc.VectorSubcoreMesh(core_axis_name="c", subcore_axis_name="s")


def _embed_body(x_ref, pos_ref, w_ref, b_ref, tab_ref):
    h = jnp.dot(x_ref[...], w_ref[...], preferred_element_type=F32) + b_ref[...]
    z = jnp.zeros((x_ref.shape[0], 93), F32)
    tab_ref[...] = jnp.concatenate([h, pos_ref[...], z], axis=1)


def _embed(x, pos, w, b2d):
    nb = _N // _NB
    return pl.pallas_call(
        _embed_body,
        grid=(nb,),
        in_specs=[
            pl.BlockSpec((_NB, 128), lambda i: (i, 0)),
            pl.BlockSpec((_NB, 3), lambda i: (i, 0)),
            pl.BlockSpec((128, _EMB), lambda i: (0, 0)),
            pl.BlockSpec((1, _EMB), lambda i: (0, 0)),
        ],
        out_specs=pl.BlockSpec((_NB, 128), lambda i: (i, 0)),
        out_shape=jax.ShapeDtypeStruct((_N, 128), F32),
    )(x, pos, w, b2d)


def _make_gather():
    @functools.partial(
        pl.kernel,
        mesh=_mesh(),
        out_type=[jax.ShapeDtypeStruct((_E, 128), F32),
                  jax.ShapeDtypeStruct((_E, 128), F32)],
        scratch_types=[
            pltpu.VMEM((_GPER,), jnp.int32),
            pltpu.VMEM((_SCH, 128), F32),
            pltpu.VMEM((_SCH, 128), F32),
            pltpu.SemaphoreType.DMA,
            pltpu.SemaphoreType.DMA,
        ],
    )
    def gk(tab, i0, i1, o0, o1, idxv, bufa, bufb, sema, semb):
        w = lax.axis_index("s") * _NC + lax.axis_index("c")
        base = w * _GPER

        for ih, oh in ((i0, o0), (i1, o1)):
            pltpu.sync_copy(ih.at[pl.ds(base, _GPER)], idxv)

            def fire(j, buf, sem):
                return pltpu.async_copy(
                    tab.at[idxv.at[pl.ds(j * _SCH, _SCH)]], buf, sem)

            def wfire(j, buf, sem):
                pltpu.make_async_copy(
                    tab.at[idxv.at[pl.ds(j * _SCH, _SCH)]], buf, sem).wait()

            fire(0, bufa, sema)
            fire(1, bufb, semb)

            def step(i, c, oh=oh):
                ja = 2 * i
                wfire(ja, bufa, sema)
                pltpu.sync_copy(bufa, oh.at[pl.ds(base + ja * _SCH, _SCH)])
                fire(ja + 2, bufa, sema)
                wfire(ja + 1, bufb, semb)
                pltpu.sync_copy(bufb, oh.at[pl.ds(base + (ja + 1) * _SCH, _SCH)])

                @pl.when(i < (_GFULL - 1) // 2 - 1)
                def _():
                    fire(ja + 3, bufb, semb)
                return c

            lax.fori_loop(0, (_GFULL - 1) // 2, step, 0)
            wfire(_GFULL - 1, bufa, sema)
            pltpu.sync_copy(bufa,
                            oh.at[pl.ds(base + (_GFULL - 1) * _SCH, _SCH)])
            pltpu.async_copy(
                tab.at[idxv.at[pl.ds(_GFULL * _SCH, _GTAIL)]],
                bufa.at[pl.ds(0, _GTAIL)], sema).wait()
            pltpu.sync_copy(bufa.at[pl.ds(0, _GTAIL)],
                            oh.at[pl.ds(base + _GFULL * _SCH, _GTAIL)])

    return gk


def _mlp_tail(ef, hs0, hs1, w1e_ref, w1a_ref, w1b_ref, b1_ref,
              w2a_ref, b2a_ref):
    base = jnp.dot(ef, w1e_ref[...], preferred_element_type=F32) + b1_ref[...]
    p0a = jnp.dot(hs0, w1a_ref[...], preferred_element_type=F32)
    p0b = jnp.dot(hs0, w1b_ref[...], preferred_element_type=F32)
    p1a = jnp.dot(hs1, w1a_ref[...], preferred_element_type=F32)
    p1b = jnp.dot(hs1, w1b_ref[...], preferred_element_type=F32)
    hf = jax.nn.relu(base + p0a + p1b)
    hr = jax.nn.relu(base + p1a + p0b)
    taf = jnp.dot(hf, w2a_ref[...], preferred_element_type=F32) + b2a_ref[...]
    tar = jnp.dot(hr, w2a_ref[...], preferred_element_type=F32) + b2a_ref[...]
    stf = jnp.concatenate([hs0, hs0, hs0], axis=1)
    str_ = jnp.concatenate([hs1, hs1, hs1], axis=1)
    return (hf, hr), (taf * stf, tar * str_)


def _pack_rec(rec_ref, a, wp, yf16, par):
    ys = (yf16, yf16 * par)
    one = jnp.ones((_EB, 16), F32)
    z32 = jnp.zeros((_EB, 32), F32)
    for ci in range(2):
        for d in range(2):
            ad = a[d]
            parts = [ad[:, 32 + ci * 16:48 + ci * 16],
                     ad[:, 64 + ci * 16:80 + ci * 16],
                     ys[d]]
            if wp is not None:
                wd = wp[d]
                parts += [wd[:, ci * 16:16 + ci * 16],
                          wd[:, 32 + ci * 16:48 + ci * 16]]
            else:
                parts.append(z32)
            parts += [ad[:, 0:32], one]
            rec_ref[ci, d] = jnp.concatenate(parts, axis=1)


def _edge1_body(g0_ref, g1_ref, w1e_ref, w1a_ref, w1b_ref, b1_ref,
                w2a_ref, b2a_ref, rec_ref, y_ref, ef_ref):
    g0 = g0_ref[...]
    g1 = g1_ref[...]
    hs0 = g0[:, 0:32]
    hs1 = g1[:, 0:32]
    dd = g0[:, 32:35] - g1[:, 32:35]
    r = jnp.sqrt(jnp.sum(dd * dd, axis=1, keepdims=True))
    vh = dd / (r + 1e-9)
    vx, vy, vz = vh[:, 0:1], vh[:, 1:2], vh[:, 2:3]
    zer7 = jnp.zeros((_EB, 7), F32)
    yf = jnp.concatenate([
        jnp.ones_like(vx), _S3 * vy, _S3 * vz, _S3 * vx,
        _S15 * vx * vy, _S15 * vy * vz, (_S5 / 2.0) * (3.0 * vz * vz - 1.0),
        _S15 * vx * vz, (_S15 / 2.0) * (vx * vx - vy * vy), zer7], axis=1)
    nrow = (lax.broadcasted_iota(jnp.int32, (1, 8), 1) + 1).astype(F32)
    bes = (2.0 / _R_MAX) ** 0.5 * jnp.sin(nrow * (np.pi / _R_MAX) * r) / (r + 1e-9)
    xr = r * (1.0 / _R_MAX)
    x2 = xr * xr
    x5 = x2 * x2 * xr
    env = 1.0 - 21.0 * x5 + 35.0 * x5 * xr - 15.0 * x5 * x2
    env = jnp.where(xr < 1.0, env, 0.0)
    ef = bes * env
    _, a = _mlp_tail(ef, hs0, hs1, w1e_ref, w1a_ref, w1b_ref, b1_ref,
                     w2a_ref, b2a_ref)
    col = lax.broadcasted_iota(jnp.int32, (1, 16), 1)
    par = jnp.where((col >= 1) & (col <= 3), -1.0, 1.0).astype(F32)
    _pack_rec(rec_ref, a, None, yf, par)
    y_ref[...] = yf
    ef_ref[...] = ef


def _edge1(g0, g1, w1e, w1a, w1b, b1, w2a, b2a):
    nb = _E // _EB
    full = lambda *s: pl.BlockSpec(s, lambda i: (0,) * len(s))
    return pl.pallas_call(
        _edge1_body,
        grid=(nb,),
        in_specs=[
            pl.BlockSpec((_EB, 128), lambda i: (i, 0)),
            pl.BlockSpec((_EB, 128), lambda i: (i, 0)),
            full(8, 256), full(32, 256), full(32, 256), full(1, 256),
            full(256, 96), full(1, 96),
        ],
        out_specs=[
            pl.BlockSpec((2, 2, _EB, 128), lambda i: (0, 0, i, 0)),
            pl.BlockSpec((_EB, 16), lambda i: (i, 0)),
            pl.BlockSpec((_EB, 8), lambda i: (i, 0)),
        ],
        out_shape=[
            jax.ShapeDtypeStruct((2, 2, _E, 128), F32),
            jax.ShapeDtypeStruct((_E, 16), F32),
            jax.ShapeDtypeStruct((_E, 8), F32),
        ],
    )(g0, g1, w1e, w1a, w1b, b1, w2a, b2a)


def _edge2_body(g0_ref, g1_ref, ef_ref, y_ref, w1e_ref, w1a_ref, w1b_ref,
                b1_ref, w2a_ref, b2a_ref, w2p_ref, b2p_ref, rec_ref):
    hs0 = g0_ref[:, 0:32]
    hs1 = g1_ref[:, 0:32]
    ef = ef_ref[...]
    yf = y_ref[...]
    hid, a = _mlp_tail(ef, hs0, hs1, w1e_ref, w1a_ref, w1b_ref, b1_ref,
                       w2a_ref, b2a_ref)
    tp = tuple(
        jnp.dot(h, w2p_ref[...], preferred_element_type=F32) + b2p_ref[...]
        for h in hid)
    col = lax.broadcasted_iota(jnp.int32, (1, 16), 1)
    par = jnp.where((col >= 1) & (col <= 3), -1.0, 1.0).astype(F32)
    _pack_rec(rec_ref, a, tp, yf, par)


def _edge2(g0, g1, ef, yx, w1e, w1a, w1b, b1, w2a, b2a, w2p, b2p):
    nb = _E // _EB
    full = lambda *s: pl.BlockSpec(s, lambda i: (0,) * len(s))
    return pl.pallas_call(
        _edge2_body,
        grid=(nb,),
        in_specs=[
            pl.BlockSpec((_EB, 128), lambda i: (i, 0)),
            pl.BlockSpec((_EB, 128), lambda i: (i, 0)),
            pl.BlockSpec((_EB, 8), lambda i: (i, 0)),
            pl.BlockSpec((_EB, 16), lambda i: (i, 0)),
            full(8, 256), full(32, 256), full(32, 256), full(1, 256),
            full(256, 96), full(1, 96),
            full(256, 64), full(1, 64),
        ],
        out_specs=pl.BlockSpec((2, 2, _EB, 128), lambda i: (0, 0, i, 0)),
        out_shape=jax.ShapeDtypeStruct((2, 2, _E, 128), F32),
    )(g0, g1, ef, yx, w1e, w1a, w1b, b1, w2a, b2a, w2p, b2p)


def _zero_fill(buf, rows, cols16):
    def z(e, c):
        for q in range(cols16):
            buf[e, q * 16:(q + 1) * 16] = jnp.zeros((16,), F32)
        return c
    lax.fori_loop(0, rows, z, 0)


def _acc_init(acc, zsrc, t):
    rows = zsrc.shape[0]
    _zero_fill(zsrc, rows, 8)
    nfull = _NROWS // rows
    for q in range(nfull):
        pltpu.sync_copy(zsrc, acc.at[pl.ds(t * _NROWS + q * rows, rows)])
    rem = _NROWS - nfull * rows
    if rem:
        pltpu.sync_copy(zsrc.at[pl.ds(0, rem)],
                        acc.at[pl.ds(t * _NROWS + nfull * rows, rem)])


def _make_scatter(has_wp):
    outs = jax.ShapeDtypeStruct((2, _NPAD, 128), F32)
    scr = [
        pltpu.VMEM_SHARED((_NPAD, 128), F32),
        pltpu.VMEM((_MCH,), jnp.int32),
        pltpu.VMEM((_MCH,), jnp.int32),
        pltpu.VMEM((_MCH,), jnp.int32),
        pltpu.VMEM((_MCH,), jnp.int32),
        pltpu.VMEM((_STAIL,), jnp.int32),
        pltpu.VMEM((_MCH, 128), F32),
        pltpu.VMEM((_MCH, 128), F32),
        pltpu.VMEM((_MCH, 128), F32),
        pltpu.VMEM((_MCH, 128), F32),
        pltpu.SemaphoreType.DMA,
        pltpu.SemaphoreType.DMA,
        pltpu.SemaphoreType.DMA,
        pltpu.SemaphoreType.DMA,
    ]
    if has_wp:
        scr += [
            pltpu.VMEM((_MCH,), jnp.int32),
            pltpu.VMEM((_MCH,), jnp.int32),
            pltpu.VMEM((_STAIL,), jnp.int32),
            pltpu.VMEM((_MCH, 128), F32),
            pltpu.VMEM((_MCH, 128), F32),
            pltpu.SemaphoreType.DMA,
            pltpu.SemaphoreType.DMA,
        ]

    def body(rec, dst, src, h0t, h1t, agg_o,
             acc, dstia, dstib, dstqa, dstqb, dstt, rva, rvb, msga, msgb,
             sla, slb, sada, sadb,
             srcia=None, srcib=None, srct=None, hva=None, hvb=None,
             sha=None, shb=None):
        ci = lax.axis_index("c")
        t = lax.axis_index("s")
        _acc_init(acc, msga, t)
        plsc.subcore_barrier()
        base = t * _SPER
        pairs = _SFULL // 2

        def fire_lin(j, di, rvx, sem, si):
            off = base + j * _MCH
            pltpu.async_copy(dst.at[pl.ds(off, _MCH)], di, sem)
            pltpu.async_copy(rec.at[ci, pl.ds(off, _MCH)], rvx, sem)
            if has_wp:
                pltpu.async_copy(src.at[pl.ds(off, _MCH)], si, sem)

        def wait_lin(j, di, rvx, sem, si):
            off = base + j * _MCH
            pltpu.make_async_copy(dst.at[pl.ds(off, _MCH)], di, sem).wait()
            pltpu.make_async_copy(rec.at[ci, pl.ds(off, _MCH)], rvx,
                                  sem).wait()
            if has_wp:
                pltpu.make_async_copy(src.at[pl.ds(off, _MCH)], si,
                                      sem).wait()

        def fire_hg(si, hvx, sem):
            @pl.when(ci == 0)
            def _():
                pltpu.async_copy(h0t.at[si], hvx, sem)

            @pl.when(ci == 1)
            def _():
                pltpu.async_copy(h1t.at[si], hvx, sem)

        def wait_hg(si, hvx, sem):
            @pl.when(ci == 0)
            def _():
                pltpu.make_async_copy(h0t.at[si], hvx, sem).wait()

            @pl.when(ci == 1)
            def _():
                pltpu.make_async_copy(h1t.at[si], hvx, sem).wait()

        def compute(k, rvx, hvx, msgx):
            def erow(e, c):
                a1 = rvx[e, 0:16]
                a2 = rvx[e, 16:32]
                yrow = rvx[e, 32:48]
                if has_wp:
                    w0 = rvx[e, 48:64]
                    w1 = rvx[e, 64:80]
                als = (None, a1, a1, a1, a2, a2, a2, a2, a2)
                for j in range(1, 9):
                    m = als[j] * yrow[j]
                    if has_wp:
                        w16 = w0 if j < 4 else w1
                        m = m + w16 * hvx[e, (j - 1) * 16:j * 16]
                    msgx[e, (j - 1) * 16:j * 16] = m
                return c

            lax.fori_loop(0, k, erow, 0)

        fire_lin(0, dstia, rva, sla, srcia)
        wait_lin(0, dstia, rva, sla, srcia)
        if has_wp:
            fire_hg(srcia, hva, sha)
        fire_lin(1, dstib, rvb, slb, srcib)

        def step(i, c):
            ja = 2 * i
            wait_lin(ja + 1, dstib, rvb, slb, srcib)
            if has_wp:
                fire_hg(srcib, hvb, shb)
                wait_hg(srcia, hva, sha)

            @pl.when(i > 0)
            def _():
                pltpu.make_async_copy(msga, acc.at[dstqa], sada).wait()

            compute(_MCH, rva, hva, msga)
            for b in range(_MCH // 16):
                dstqa[b * 16:(b + 1) * 16] = dstia[b * 16:(b + 1) * 16]
            pltpu.async_copy(msga, acc.at[dstqa], sada, add=True)

            @pl.when(i < pairs - 1)
            def _():
                fire_lin(ja + 2, dstia, rva, sla, srcia)
                wait_lin(ja + 2, dstia, rva, sla, srcia)
                if has_wp:
                    fire_hg(srcia, hva, sha)

            if has_wp:
                wait_hg(srcib, hvb, shb)

            @pl.when(i > 0)
            def _():
                pltpu.make_async_copy(msgb, acc.at[dstqb], sadb).wait()

            compute(_MCH, rvb, hvb, msgb)
            for b in range(_MCH // 16):
                dstqb[b * 16:(b + 1) * 16] = dstib[b * 16:(b + 1) * 16]
            pltpu.async_copy(msgb, acc.at[dstqb], sadb, add=True)

            @pl.when(i < pairs - 1)
            def _():
                fire_lin(ja + 3, dstib, rvb, slb, srcib)
            return c

        lax.fori_loop(0, pairs, step, 0)
        pltpu.make_async_copy(msga, acc.at[dstqa], sada).wait()
        pltpu.make_async_copy(msgb, acc.at[dstqb], sadb).wait()

        toff = base + _SFULL * _MCH
        pltpu.sync_copy(dst.at[pl.ds(toff, _STAIL)], dstt)
        pltpu.sync_copy(rec.at[ci, pl.ds(toff, _STAIL)],
                        rva.at[pl.ds(0, _STAIL)])
        if has_wp:
            pltpu.sync_copy(src.at[pl.ds(toff, _STAIL)], srct)

            @pl.when(ci == 0)
            def _():
                pltpu.async_copy(h0t.at[srct], hva.at[pl.ds(0, _STAIL)],
                                 sha).wait()

            @pl.when(ci == 1)
            def _():
                pltpu.async_copy(h1t.at[srct], hva.at[pl.ds(0, _STAIL)],
                                 sha).wait()

        def erow_t(e, c):
            a1 = rva[e, 0:16]
            a2 = rva[e, 16:32]
            yrow = rva[e, 32:48]
            if has_wp:
                w0 = rva[e, 48:64]
                w1 = rva[e, 64:80]
            als = (None, a1, a1, a1, a2, a2, a2, a2, a2)
            for j in range(1, 9):
                m = als[j] * yrow[j]
                if has_wp:
                    w16 = w0 if j < 4 else w1
                    m = m + w16 * hva[e, (j - 1) * 16:j * 16]
                msga[e, (j - 1) * 16:j * 16] = m
            return c

        lax.fori_loop(0, _STAIL, erow_t, 0)
        pltpu.sync_copy(msga.at[pl.ds(0, _STAIL)], acc.at[dstt], add=True)

        plsc.subcore_barrier()
        pltpu.sync_copy(acc.at[pl.ds(t * _NROWS, _NROWS)],
                        agg_o.at[ci, pl.ds(t * _NROWS, _NROWS)])

    if has_wp:
        @functools.partial(pl.kernel, mesh=_mesh(), out_type=outs,
                           scratch_types=scr)
        def sk(rec, dst, src, h0t, h1t, agg_o,
               acc, dstia, dstib, dstqa, dstqb, dstt, rva, rvb, msga, msgb,
               sla, slb, sada, sadb, srcia, srcib, srct, hva, hvb, sha, shb):
            body(rec, dst, src, h0t, h1t, agg_o,
                 acc, dstia, dstib, dstqa, dstqb, dstt, rva, rvb, msga, msgb,
                 sla, slb, sada, sadb, srcia, srcib, srct, hva, hvb, sha, shb)
    else:
        @functools.partial(pl.kernel, mesh=_mesh(), out_type=outs,
                           scratch_types=scr)
        def sk(rec, dst, agg_o,
               acc, dstia, dstib, dstqa, dstqb, dstt, rva, rvb, msga, msgb,
               sla, slb, sada, sadb):
            body(rec, dst, None, None, None, agg_o,
                 acc, dstia, dstib, dstqa, dstqb, dstt, rva, rvb, msga, msgb,
                 sla, slb, sada, sadb)
    return sk


def _make_scatter0():
    outs = jax.ShapeDtypeStruct((2, _NPAD, 128), F32)
    scr = [
        pltpu.VMEM_SHARED((_NPAD, 128), F32),
        pltpu.VMEM((_SCH,), jnp.int32),
        pltpu.VMEM((_SCH,), jnp.int32),
        pltpu.VMEM((_PTAIL,), jnp.int32),
        pltpu.VMEM((_SCH, 128), F32),
        pltpu.VMEM((_SCH, 128), F32),
        pltpu.SemaphoreType.DMA,
        pltpu.SemaphoreType.DMA,
    ]

    @functools.partial(pl.kernel, mesh=_mesh(), out_type=outs,
                       scratch_types=scr)
    def sk(pay, dst, agg_o, acc, dstia, dstib, dstt, pva, pvb, sema, semb):
        ci = lax.axis_index("c")
        t = lax.axis_index("s")
        _acc_init(acc, pva, t)
        plsc.subcore_barrier()
        base = (t * _NC + ci) * _PPER
        pairs = _PFULL // 2

        def fire(j, di, pv, sem):
            off = base + j * _SCH
            pltpu.async_copy(dst.at[pl.ds(off, _SCH)], di, sem)
            pltpu.async_copy(pay.at[ci, pl.ds(off, _SCH)], pv, sem)

        def wfire(j, di, pv, sem):
            off = base + j * _SCH
            pltpu.make_async_copy(dst.at[pl.ds(off, _SCH)], di, sem).wait()
            pltpu.make_async_copy(pay.at[ci, pl.ds(off, _SCH)], pv,
                                  sem).wait()

        fire(0, dstia, pva, sema)
        fire(1, dstib, pvb, semb)

        def step(i, c):
            ja = 2 * i
            wfire(ja, dstia, pva, sema)
            pltpu.sync_copy(pva, acc.at[dstia], add=True)

            @pl.when(i < pairs - 1)
            def _():
                fire(ja + 2, dstia, pva, sema)

            wfire(ja + 1, dstib, pvb, semb)
            pltpu.sync_copy(pvb, acc.at[dstib], add=True)

            @pl.when(i < pairs - 1)
            def _():
                fire(ja + 3, dstib, pvb, semb)
            return c

        lax.fori_loop(0, pairs, step, 0)

        toff = base + _PFULL * _SCH
        pltpu.sync_copy(dst.at[pl.ds(toff, _PTAIL)], dstt)
        pltpu.sync_copy(pay.at[ci, pl.ds(toff, _PTAIL)],
                        pva.at[pl.ds(0, _PTAIL)])
        pltpu.sync_copy(pva.at[pl.ds(0, _PTAIL)], acc.at[dstt], add=True)

        plsc.subcore_barrier()
        pltpu.sync_copy(acc.at[pl.ds(t * _NROWS, _NROWS)],
                        agg_o.at[ci, pl.ds(t * _NROWS, _NROWS)])

    return sk


def _nodeP1_body(agg_ref, aggs_ref, lin_ref, o_ref, st_ref):
    i = pl.program_id(0)
    cnt = aggs_ref[0, :, 112:113] + aggs_ref[1, :, 112:113]
    inv = 1.0 / jnp.maximum(cnt, 1.0)
    os_ = []
    for k in range(9):
        if k == 0:
            xk = (aggs_ref[0, :, 80:112] + aggs_ref[1, :, 80:112]) * inv
        else:
            xk = jnp.concatenate(
                [agg_ref[0, :, (k - 1) * 16:k * 16],
                 agg_ref[1, :, (k - 1) * 16:k * 16]], axis=1) * inv
        os_.append(jnp.dot(xk, lin_ref[_LMAP[k]], preferred_element_type=F32))
    o_ref[...] = jnp.concatenate(os_, axis=1)
    s = os_[0]
    n1 = jnp.sqrt(os_[1] ** 2 + os_[2] ** 2 + os_[3] ** 2 + 1e-12)
    n2 = jnp.sqrt(os_[4] ** 2 + os_[5] ** 2 + os_[6] ** 2
                  + os_[7] ** 2 + os_[8] ** 2 + 1e-12)
    part = jnp.concatenate([
        jnp.sum(s, axis=0, keepdims=True),
        jnp.sum(s * s, axis=0, keepdims=True),
        jnp.sum(n1, axis=0, keepdims=True),
        jnp.sum(n2, axis=0, keepdims=True)], axis=0)

    @pl.when(i == 0)
    def _():
        st_ref[...] = part

    @pl.when(i != 0)
    def _():
        st_ref[...] = st_ref[...] + part


def _nodeP1(agg, aggs, lin):
    nb = _N // _NB
    full = lambda *s: pl.BlockSpec(s, lambda i: (0,) * len(s))
    return pl.pallas_call(
        _nodeP1_body,
        grid=(nb,),
        in_specs=[
            pl.BlockSpec((2, _NB, 128), lambda i: (0, i, 0)),
            pl.BlockSpec((2, _NB, 128), lambda i: (0, i, 0)),
            full(3, 32, 32),
        ],
        out_specs=[
            pl.BlockSpec((_NB, 288), lambda i: (i, 0)),
            full(4, 32),
        ],
        out_shape=[
            jax.ShapeDtypeStruct((_N, 288), F32),
            jax.ShapeDtypeStruct((4, 32), F32),
        ],
    )(agg, aggs, lin)


def _bn_apply(o_ref, st_ref):
    mu = st_ref[0:1, :] * (1.0 / _N)
    var = st_ref[1:2, :] * (1.0 / _N) - mu * mu
    sd = jnp.sqrt(jnp.maximum(var, 0.0))
    isd = 1.0 / (sd + 1e-5)
    in1 = 1.0 / (st_ref[2:3, :] * (1.0 / _N) + 1e-5)
    in2 = 1.0 / (st_ref[3:4, :] * (1.0 / _N) + 1e-5)
    aks = []
    for k in range(9):
        pk = o_ref[:, k * 32:(k + 1) * 32]
        if k == 0:
            aks.append((pk - mu) * isd)
        elif k < 4:
            aks.append(pk * in1)
        else:
            aks.append(pk * in2)
    return aks


def _nodeP2_body(o_ref, st_ref, pw_ref, plin_ref, o2_ref, st2_ref):
    i = pl.program_id(0)
    aks = _bn_apply(o_ref, st_ref)
    a0 = aks[0]
    f = pw_ref[0:1, :] + pw_ref[1:2, :] * a0 + pw_ref[2:3, :] * a0 * a0
    ps = [jnp.dot(f * aks[k], plin_ref[_LMAP[k]], preferred_element_type=F32)
          for k in range(9)]
    o2_ref[...] = jnp.concatenate(ps, axis=1)
    s = ps[0]
    n1 = jnp.sqrt(ps[1] ** 2 + ps[2] ** 2 + ps[3] ** 2 + 1e-12)
    n2 = jnp.sqrt(ps[4] ** 2 + ps[5] ** 2 + ps[6] ** 2
                  + ps[7] ** 2 + ps[8] ** 2 + 1e-12)
    part = jnp.concatenate([
        jnp.sum(s, axis=0, keepdims=True),
        jnp.sum(s * s, axis=0, keepdims=True),
        jnp.sum(n1, axis=0, keepdims=True),
        jnp.sum(n2, axis=0, keepdims=True)], axis=0)

    @pl.when(i == 0)
    def _():
        st2_ref[...] = part

    @pl.when(i != 0)
    def _():
        st2_ref[...] = st2_ref[...] + part


def _nodeP2(o1, st1, pw2d, plin):
    nb = _N // _NB
    full = lambda *s: pl.BlockSpec(s, lambda i: (0,) * len(s))
    return pl.pallas_call(
        _nodeP2_body,
        grid=(nb,),
        in_specs=[
            pl.BlockSpec((_NB, 288), lambda i: (i, 0)),
            full(4, 32), full(3, 32), full(3, 32, 32),
        ],
        out_specs=[
            pl.BlockSpec((_NB, 288), lambda i: (i, 0)),
            full(4, 32),
        ],
        out_shape=[
            jax.ShapeDtypeStruct((_N, 288), F32),
            jax.ShapeDtypeStruct((4, 32), F32),
        ],
    )(o1, st1, pw2d, plin)


def _nodeP3_body(o2_ref, st_ref, h0_ref, hs2_ref, hsc_ref):
    bks = _bn_apply(o2_ref, st_ref)
    b0 = bks[0] + h0_ref[:, 0:32]
    hs2_ref[...] = jnp.concatenate(
        [b0, jnp.zeros((b0.shape[0], 96), F32)], axis=1)
    for ci in range(2):
        hsc_ref[ci] = jnp.concatenate(
            [bks[k][:, ci * 16:(ci + 1) * 16] for k in range(1, 9)], axis=1)


def _nodeP3(o2, st2, tab):
    nb = _N // _NB
    full = lambda *s: pl.BlockSpec(s, lambda i: (0,) * len(s))
    return pl.pallas_call(
        _nodeP3_body,
        grid=(nb,),
        in_specs=[
            pl.BlockSpec((_NB, 288), lambda i: (i, 0)),
            full(4, 32),
            pl.BlockSpec((_NB, 128), lambda i: (i, 0)),
        ],
        out_specs=[
            pl.BlockSpec((_NB, 128), lambda i: (i, 0)),
            pl.BlockSpec((2, _NB, 128), lambda i: (0, i, 0)),
        ],
        out_shape=[
            jax.ShapeDtypeStruct((_N, 128), F32),
            jax.ShapeDtypeStruct((2, _N, 128), F32),
        ],
    )(o2, st2, tab)


def _nodeQ_body(o_ref, st_ref, pw_ref, plin_ref, q_ref, st2_ref):
    i = pl.program_id(0)
    aks = _bn_apply(o_ref, st_ref)
    a0 = aks[0]
    f = pw_ref[0:1, :] + pw_ref[1:2, :] * a0 + pw_ref[2:3, :] * a0 * a0
    q = jnp.dot(f * a0, plin_ref[0], preferred_element_type=F32)
    q_ref[...] = q
    part = jnp.concatenate([
        jnp.sum(q, axis=0, keepdims=True),
        jnp.sum(q * q, axis=0, keepdims=True)], axis=0)

    @pl.when(i == 0)
    def _():
        st2_ref[...] = part

    @pl.when(i != 0)
    def _():
        st2_ref[...] = st2_ref[...] + part


def _nodeQ(o1, st1, pw2d, plin):
    nb = _N // _NB
    full = lambda *s: pl.BlockSpec(s, lambda i: (0,) * len(s))
    return pl.pallas_call(
        _nodeQ_body,
        grid=(nb,),
        in_specs=[
            pl.BlockSpec((_NB, 288), lambda i: (i, 0)),
            full(4, 32), full(3, 32), full(3, 32, 32),
        ],
        out_specs=[
            pl.BlockSpec((_NB, 32), lambda i: (i, 0)),
            full(2, 32),
        ],
        out_shape=[
            jax.ShapeDtypeStruct((_N, 32), F32),
            jax.ShapeDtypeStruct((2, 32), F32),
        ],
    )(o1, st1, pw2d, plin)


def _nodeF_body(q_ref, st_ref, b_ref, node_ref, graph_ref):
    i = pl.program_id(0)
    mu = st_ref[0:1, :] * (1.0 / _N)
    var = st_ref[1:2, :] * (1.0 / _N) - mu * mu
    sd = jnp.sqrt(jnp.maximum(var, 0.0))
    node = (q_ref[...] - mu) / (sd + 1e-5)
    node_ref[...] = node
    bt = b_ref[0, 0, :]
    oht = (lax.broadcasted_iota(jnp.int32, (64, _NB), 0)
           == bt[None, :]).astype(F32)
    g = jnp.dot(oht, node, preferred_element_type=F32)

    @pl.when(i == 0)
    def _():
        graph_ref[...] = g

    @pl.when(i != 0)
    def _():
        graph_ref[...] = graph_ref[...] + g


def _nodeF(q, st3, batch3):
    nb = _N // _NB
    full = lambda *s: pl.BlockSpec(s, lambda i: (0,) * len(s))
    return pl.pallas_call(
        _nodeF_body,
        grid=(nb,),
        in_specs=[
            pl.BlockSpec((_NB, 32), lambda i: (i, 0)),
            full(2, 32),
            pl.BlockSpec((1, 1, _NB), lambda i: (i, 0, 0)),
        ],
        out_specs=[
            pl.BlockSpec((_NB, 32), lambda i: (i, 0)),
            full(64, 32),
        ],
        out_shape=[
            jax.ShapeDtypeStruct((_N, 32), F32),
            jax.ShapeDtypeStruct((64, 32), F32),
        ],
    )(q, st3, batch3)


def _perm_ws():
    return np.array([3 * c + l for l in range(3) for c in range(32)])


def _perm_wp():
    return np.array([2 * c + m for m in range(2) for c in range(32)])


_gather128 = _make_gather()
_scatter1 = _make_scatter(False)
_scatter2 = _make_scatter(True)
_scatter0 = _make_scatter0()


def kernel(x, pos, edge_index, batch, W_emb, b_emb, mW1_0, mb1_0, mW2_0,
           mb2_0, lin_0, pw_0, plin_0, mW1_1, mb1_1, mW2_1, mb2_1, lin_1,
           pw_1, plin_1):
    e0 = edge_index[0].astype(jnp.int32)
    e1 = edge_index[1].astype(jnp.int32)
    src_all = jnp.concatenate([e0, e1])
    dst_all = jnp.concatenate([e1, e0])

    pws = _perm_ws()
    pwp = _perm_wp()
    w1e_0, w1a_0, w1b_0 = mW1_0[0:8], mW1_0[8:40], mW1_0[40:72]
    w1e_1, w1a_1, w1b_1 = mW1_1[0:8], mW1_1[8:40], mW1_1[40:72]
    w2a_0 = mW2_0[:, pws]
    b2a_0 = mb2_0[pws][None, :]
    w2a_1 = mW2_1[:, :96][:, pws]
    b2a_1 = mb2_1[:96][pws][None, :]
    w2p_1 = mW2_1[:, 96:160][:, pwp]
    b2p_1 = mb2_1[96:160][pwp][None, :]

    tab = _embed(x, pos, W_emb, b_emb[None, :])
    g0, g1 = _gather128(tab, e0, e1)
    rec1, yx, ef = _edge1(g0, g1, w1e_0, w1a_0, w1b_0, mb1_0[None, :],
                          w2a_0, b2a_0)
    rec1 = rec1.reshape(2, _E2, 128)
    agg1 = _scatter1(rec1, dst_all)
    aggs1 = _scatter0(rec1, dst_all)
    o1, st1 = _nodeP1(agg1, aggs1, lin_0)
    o2, st2 = _nodeP2(o1, st1, pw_0, plin_0)
    hs2, h2sc = _nodeP3(o2, st2, tab)

    g20, g21 = _gather128(hs2, e0, e1)
    rec2 = _edge2(g20, g21, ef, yx, w1e_1, w1a_1, w1b_1,
                  mb1_1[None, :], w2a_1, b2a_1, w2p_1, b2p_1)
    rec2 = rec2.reshape(2, _E2, 128)
    agg2 = _scatter2(rec2, dst_all, src_all, h2sc[0], h2sc[1])
    aggs2 = _scatter0(rec2, dst_all)
    o1b, st1b = _nodeP1(agg2, aggs2, lin_1)
    q, st3 = _nodeQ(o1b, st1b, pw_1, plin_1)
    node, graph = _nodeF(q, st3,
                         batch.astype(jnp.int32).reshape(_N // _NB, 1, _NB))
    return (node, graph)


# speedup vs baseline: 1.0625x; 1.0625x over previous
import functools

import jax
import jax.numpy as jnp
import numpy as np
from jax import lax
from jax.experimental import pallas as pl
from jax.experimental.pallas import tpu as pltpu
from jax.experimental.pallas import tpu_sc as plsc

F32 = jnp.float32
_N = 10000
_E = 160000
_E2 = 2 * _E
_EMB = 32
_R_MAX = 10.0
_NB = 1000
_EB = 1000
_LMAP = [0, 1, 1, 1, 2, 2, 2, 2, 2]
_S3 = 3.0 ** 0.5
_S15 = 15.0 ** 0.5
_S5 = 5.0 ** 0.5

_NC, _NS = 2, 16
_NW = _NC * _NS
_GPER = _E // _NW
_SPER = _E2 // _NS
_SCH = 128
_GFULL = _GPER // _SCH
_GTAIL = _GPER - _GFULL * _SCH
_MCH = 64
_STAIL = 32
_SFULL = (_SPER - _STAIL) // _MCH
_PPER = _E2 // _NW
_PFULL = _PPER // _SCH
_PTAIL = _PPER - _PFULL * _SCH
_NPAD = 10112
_NROWS = _NPAD // _NS


def _mesh():
    return plsc.VectorSubcoreMesh(core_axis_name="c", subcore_axis_name="s")


def _embed_body(x_ref, pos_ref, w_ref, b_ref, tab_ref):
    h = jnp.dot(x_ref[...], w_ref[...], preferred_element_type=F32) + b_ref[...]
    z = jnp.zeros((x_ref.shape[0], 93), F32)
    tab_ref[...] = jnp.concatenate([h, pos_ref[...], z], axis=1)


def _embed(x, pos, w, b2d):
    nb = _N // _NB
    return pl.pallas_call(
        _embed_body,
        grid=(nb,),
        in_specs=[
            pl.BlockSpec((_NB, 128), lambda i: (i, 0)),
            pl.BlockSpec((_NB, 3), lambda i: (i, 0)),
            pl.BlockSpec((128, _EMB), lambda i: (0, 0)),
            pl.BlockSpec((1, _EMB), lambda i: (0, 0)),
        ],
        out_specs=pl.BlockSpec((_NB, 128), lambda i: (i, 0)),
        out_shape=jax.ShapeDtypeStruct((_N, 128), F32),
    )(x, pos, w, b2d)


def _make_gather():
    @functools.partial(
        pl.kernel,
        mesh=_mesh(),
        out_type=[jax.ShapeDtypeStruct((_E, 128), F32),
                  jax.ShapeDtypeStruct((_E, 128), F32)],
        scratch_types=[
            pltpu.VMEM((_GPER,), jnp.int32),
            pltpu.VMEM((_SCH, 128), F32),
            pltpu.VMEM((_SCH, 128), F32),
            pltpu.SemaphoreType.DMA,
            pltpu.SemaphoreType.DMA,
        ],
    )
    def gk(tab, i0, i1, o0, o1, idxv, bufa, bufb, sema, semb):
        w = lax.axis_index("s") * _NC + lax.axis_index("c")
        base = w * _GPER

        for ih, oh in ((i0, o0), (i1, o1)):
            pltpu.sync_copy(ih.at[pl.ds(base, _GPER)], idxv)

            def fire(j, buf, sem):
                return pltpu.async_copy(
                    tab.at[idxv.at[pl.ds(j * _SCH, _SCH)]], buf, sem)

            def wfire(j, buf, sem):
                pltpu.make_async_copy(
                    tab.at[idxv.at[pl.ds(j * _SCH, _SCH)]], buf, sem).wait()

            fire(0, bufa, sema)
            fire(1, bufb, semb)

            def step(i, c, oh=oh):
                ja = 2 * i
                wfire(ja, bufa, sema)
                pltpu.sync_copy(bufa, oh.at[pl.ds(base + ja * _SCH, _SCH)])
                fire(ja + 2, bufa, sema)
                wfire(ja + 1, bufb, semb)
                pltpu.sync_copy(bufb, oh.at[pl.ds(base + (ja + 1) * _SCH, _SCH)])

                @pl.when(i < (_GFULL - 1) // 2 - 1)
                def _():
                    fire(ja + 3, bufb, semb)
                return c

            lax.fori_loop(0, (_GFULL - 1) // 2, step, 0)
            wfire(_GFULL - 1, bufa, sema)
            pltpu.sync_copy(bufa,
                            oh.at[pl.ds(base + (_GFULL - 1) * _SCH, _SCH)])
            pltpu.async_copy(
                tab.at[idxv.at[pl.ds(_GFULL * _SCH, _GTAIL)]],
                bufa.at[pl.ds(0, _GTAIL)], sema).wait()
            pltpu.sync_copy(bufa.at[pl.ds(0, _GTAIL)],
                            oh.at[pl.ds(base + _GFULL * _SCH, _GTAIL)])

    return gk


def _mlp_tail(ef, hs0, hs1, w1e_ref, w1a_ref, w1b_ref, b1_ref,
              w2a_ref, b2a_ref):
    base = jnp.dot(ef, w1e_ref[...], preferred_element_type=F32) + b1_ref[...]
    p0a = jnp.dot(hs0, w1a_ref[...], preferred_element_type=F32)
    p0b = jnp.dot(hs0, w1b_ref[...], preferred_element_type=F32)
    p1a = jnp.dot(hs1, w1a_ref[...], preferred_element_type=F32)
    p1b = jnp.dot(hs1, w1b_ref[...], preferred_element_type=F32)
    hf = jax.nn.relu(base + p0a + p1b)
    hr = jax.nn.relu(base + p1a + p0b)
    taf = jnp.dot(hf, w2a_ref[...], preferred_element_type=F32) + b2a_ref[...]
    tar = jnp.dot(hr, w2a_ref[...], preferred_element_type=F32) + b2a_ref[...]
    stf = jnp.concatenate([hs0, hs0, hs0], axis=1)
    str_ = jnp.concatenate([hs1, hs1, hs1], axis=1)
    return (hf, hr), (taf * stf, tar * str_)


def _pack_rec(rec_ref, a, wp, yf16, par):
    ys = (yf16, yf16 * par)
    one = jnp.ones((_EB, 16), F32)
    z32 = jnp.zeros((_EB, 32), F32)
    for ci in range(2):
        for d in range(2):
            ad = a[d]
            parts = [ad[:, 32 + ci * 16:48 + ci * 16],
                     ad[:, 64 + ci * 16:80 + ci * 16],
                     ys[d]]
            if wp is not None:
                wd = wp[d]
                parts += [wd[:, ci * 16:16 + ci * 16],
                          wd[:, 32 + ci * 16:48 + ci * 16]]
            else:
                parts.append(z32)
            parts += [ad[:, 0:32], one]
            rec_ref[ci, d] = jnp.concatenate(parts, axis=1)


def _edge1_body(g0_ref, g1_ref, w1e_ref, w1a_ref, w1b_ref, b1_ref,
                w2a_ref, b2a_ref, rec_ref, y_ref, ef_ref):
    g0 = g0_ref[...]
    g1 = g1_ref[...]
    hs0 = g0[:, 0:32]
    hs1 = g1[:, 0:32]
    dd = g0[:, 32:35] - g1[:, 32:35]
    r = jnp.sqrt(jnp.sum(dd * dd, axis=1, keepdims=True))
    vh = dd / (r + 1e-9)
    vx, vy, vz = vh[:, 0:1], vh[:, 1:2], vh[:, 2:3]
    zer7 = jnp.zeros((_EB, 7), F32)
    yf = jnp.concatenate([
        jnp.ones_like(vx), _S3 * vy, _S3 * vz, _S3 * vx,
        _S15 * vx * vy, _S15 * vy * vz, (_S5 / 2.0) * (3.0 * vz * vz - 1.0),
        _S15 * vx * vz, (_S15 / 2.0) * (vx * vx - vy * vy), zer7], axis=1)
    nrow = (lax.broadcasted_iota(jnp.int32, (1, 8), 1) + 1).astype(F32)
    bes = (2.0 / _R_MAX) ** 0.5 * jnp.sin(nrow * (np.pi / _R_MAX) * r) / (r + 1e-9)
    xr = r * (1.0 / _R_MAX)
    x2 = xr * xr
    x5 = x2 * x2 * xr
    env = 1.0 - 21.0 * x5 + 35.0 * x5 * xr - 15.0 * x5 * x2
    env = jnp.where(xr < 1.0, env, 0.0)
    ef = bes * env
    _, a = _mlp_tail(ef, hs0, hs1, w1e_ref, w1a_ref, w1b_ref, b1_ref,
                     w2a_ref, b2a_ref)
    col = lax.broadcasted_iota(jnp.int32, (1, 16), 1)
    par = jnp.where((col >= 1) & (col <= 3), -1.0, 1.0).astype(F32)
    _pack_rec(rec_ref, a, None, yf, par)
    y_ref[...] = yf
    ef_ref[...] = ef


def _edge1(g0, g1, w1e, w1a, w1b, b1, w2a, b2a):
    nb = _E // _EB
    full = lambda *s: pl.BlockSpec(s, lambda i: (0,) * len(s))
    return pl.pallas_call(
        _edge1_body,
        grid=(nb,),
        in_specs=[
            pl.BlockSpec((_EB, 128), lambda i: (i, 0)),
            pl.BlockSpec((_EB, 128), lambda i: (i, 0)),
            full(8, 256), full(32, 256), full(32, 256), full(1, 256),
            full(256, 96), full(1, 96),
        ],
        out_specs=[
            pl.BlockSpec((2, 2, _EB, 128), lambda i: (0, 0, i, 0)),
            pl.BlockSpec((_EB, 16), lambda i: (i, 0)),
            pl.BlockSpec((_EB, 8), lambda i: (i, 0)),
        ],
        out_shape=[
            jax.ShapeDtypeStruct((2, 2, _E, 128), F32),
            jax.ShapeDtypeStruct((_E, 16), F32),
            jax.ShapeDtypeStruct((_E, 8), F32),
        ],
    )(g0, g1, w1e, w1a, w1b, b1, w2a, b2a)


def _edge2_body(g0_ref, g1_ref, ef_ref, y_ref, w1e_ref, w1a_ref, w1b_ref,
                b1_ref, w2a_ref, b2a_ref, w2p_ref, b2p_ref, rec_ref):
    hs0 = g0_ref[:, 0:32]
    hs1 = g1_ref[:, 0:32]
    ef = ef_ref[...]
    yf = y_ref[...]
    hid, a = _mlp_tail(ef, hs0, hs1, w1e_ref, w1a_ref, w1b_ref, b1_ref,
                       w2a_ref, b2a_ref)
    tp = tuple(
        jnp.dot(h, w2p_ref[...], preferred_element_type=F32) + b2p_ref[...]
        for h in hid)
    col = lax.broadcasted_iota(jnp.int32, (1, 16), 1)
    par = jnp.where((col >= 1) & (col <= 3), -1.0, 1.0).astype(F32)
    _pack_rec(rec_ref, a, tp, yf, par)


def _edge2(g0, g1, ef, yx, w1e, w1a, w1b, b1, w2a, b2a, w2p, b2p):
    nb = _E // _EB
    full = lambda *s: pl.BlockSpec(s, lambda i: (0,) * len(s))
    return pl.pallas_call(
        _edge2_body,
        grid=(nb,),
        in_specs=[
            pl.BlockSpec((_EB, 128), lambda i: (i, 0)),
            pl.BlockSpec((_EB, 128), lambda i: (i, 0)),
            pl.BlockSpec((_EB, 8), lambda i: (i, 0)),
            pl.BlockSpec((_EB, 16), lambda i: (i, 0)),
            full(8, 256), full(32, 256), full(32, 256), full(1, 256),
            full(256, 96), full(1, 96),
            full(256, 64), full(1, 64),
        ],
        out_specs=pl.BlockSpec((2, 2, _EB, 128), lambda i: (0, 0, i, 0)),
        out_shape=jax.ShapeDtypeStruct((2, 2, _E, 128), F32),
    )(g0, g1, ef, yx, w1e, w1a, w1b, b1, w2a, b2a, w2p, b2p)


def _zero_fill(buf, rows, cols16):
    def z(e, c):
        for q in range(cols16):
            buf[e, q * 16:(q + 1) * 16] = jnp.zeros((16,), F32)
        return c
    lax.fori_loop(0, rows, z, 0)


def _acc_init(acc, zsrc, t):
    rows = zsrc.shape[0]
    _zero_fill(zsrc, rows, 8)
    nfull = _NROWS // rows
    for q in range(nfull):
        pltpu.sync_copy(zsrc, acc.at[pl.ds(t * _NROWS + q * rows, rows)])
    rem = _NROWS - nfull * rows
    if rem:
        pltpu.sync_copy(zsrc.at[pl.ds(0, rem)],
                        acc.at[pl.ds(t * _NROWS + nfull * rows, rem)])


def _make_scatter(has_wp):
    outs = jax.ShapeDtypeStruct((2, _NPAD, 128), F32)
    scr = [
        pltpu.VMEM_SHARED((_NPAD, 128), F32),
        pltpu.VMEM((_MCH,), jnp.int32),
        pltpu.VMEM((_MCH,), jnp.int32),
        pltpu.VMEM((_MCH,), jnp.int32),
        pltpu.VMEM((_MCH,), jnp.int32),
        pltpu.VMEM((_STAIL,), jnp.int32),
        pltpu.VMEM((_MCH, 128), F32),
        pltpu.VMEM((_MCH, 128), F32),
        pltpu.VMEM((_MCH, 128), F32),
        pltpu.VMEM((_MCH, 128), F32),
        pltpu.SemaphoreType.DMA,
        pltpu.SemaphoreType.DMA,
        pltpu.SemaphoreType.DMA,
        pltpu.SemaphoreType.DMA,
    ]
    if has_wp:
        scr += [
            pltpu.VMEM((_MCH,), jnp.int32),
            pltpu.VMEM((_MCH,), jnp.int32),
            pltpu.VMEM((_STAIL,), jnp.int32),
            pltpu.VMEM((_MCH, 128), F32),
            pltpu.VMEM((_MCH, 128), F32),
            pltpu.SemaphoreType.DMA,
            pltpu.SemaphoreType.DMA,
        ]

    def body(rec, dst, src, h0t, h1t, agg_o,
             acc, dstia, dstib, dstqa, dstqb, dstt, rva, rvb, msga, msgb,
             sla, slb, sada, sadb,
             srcia=None, srcib=None, srct=None, hva=None, hvb=None,
             sha=None, shb=None):
        ci = lax.axis_index("c")
        t = lax.axis_index("s")
        _acc_init(acc, msga, t)
        plsc.subcore_barrier()
        base = t * _SPER
        pairs = _SFULL // 2

        def fire_lin(j, di, rvx, sem, si):
            off = base + j * _MCH
            pltpu.async_copy(dst.at[pl.ds(off, _MCH)], di, sem)
            pltpu.async_copy(rec.at[ci, pl.ds(off, _MCH)], rvx, sem)
            if has_wp:
                pltpu.async_copy(src.at[pl.ds(off, _MCH)], si, sem)

        def wait_lin(j, di, rvx, sem, si):
            off = base + j * _MCH
            pltpu.make_async_copy(dst.at[pl.ds(off, _MCH)], di, sem).wait()
            pltpu.make_async_copy(rec.at[ci, pl.ds(off, _MCH)], rvx,
                                  sem).wait()
            if has_wp:
                pltpu.make_async_copy(src.at[pl.ds(off, _MCH)], si,
                                      sem).wait()

        def fire_hg(si, hvx, sem):
            @pl.when(ci == 0)
            def _():
                pltpu.async_copy(h0t.at[si], hvx, sem)

            @pl.when(ci == 1)
            def _():
                pltpu.async_copy(h1t.at[si], hvx, sem)

        def wait_hg(si, hvx, sem):
            @pl.when(ci == 0)
            def _():
                pltpu.make_async_copy(h0t.at[si], hvx, sem).wait()

            @pl.when(ci == 1)
            def _():
                pltpu.make_async_copy(h1t.at[si], hvx, sem).wait()

        def compute(k, rvx, hvx, msgx):
            @functools.partial(plsc.parallel_loop, 0, k, unroll=2)
            def erow(e):
                a1 = rvx[e, 0:16]
                a2 = rvx[e, 16:32]
                yrow = rvx[e, 32:48]
                if has_wp:
                    w0 = rvx[e, 48:64]
                    w1 = rvx[e, 64:80]
                als = (None, a1, a1, a1, a2, a2, a2, a2, a2)
                for j in range(1, 9):
                    m = als[j] * yrow[j]
                    if has_wp:
                        w16 = w0 if j < 4 else w1
                        m = m + w16 * hvx[e, (j - 1) * 16:j * 16]
                    msgx[e, (j - 1) * 16:j * 16] = m

        fire_lin(0, dstia, rva, sla, srcia)
        wait_lin(0, dstia, rva, sla, srcia)
        if has_wp:
            fire_hg(srcia, hva, sha)
        fire_lin(1, dstib, rvb, slb, srcib)

        def step(i, c):
            ja = 2 * i
            wait_lin(ja + 1, dstib, rvb, slb, srcib)
            if has_wp:
                fire_hg(srcib, hvb, shb)
                wait_hg(srcia, hva, sha)

            @pl.when(i > 0)
            def _():
                pltpu.make_async_copy(msga, acc.at[dstqa], sada).wait()

            compute(_MCH, rva, hva, msga)
            for b in range(_MCH // 16):
                dstqa[b * 16:(b + 1) * 16] = dstia[b * 16:(b + 1) * 16]
            pltpu.async_copy(msga, acc.at[dstqa], sada, add=True)

            @pl.when(i < pairs - 1)
            def _():
                fire_lin(ja + 2, dstia, rva, sla, srcia)
                wait_lin(ja + 2, dstia, rva, sla, srcia)
                if has_wp:
                    fire_hg(srcia, hva, sha)

            if has_wp:
                wait_hg(srcib, hvb, shb)

            @pl.when(i > 0)
            def _():
                pltpu.make_async_copy(msgb, acc.at[dstqb], sadb).wait()

            compute(_MCH, rvb, hvb, msgb)
            for b in range(_MCH // 16):
                dstqb[b * 16:(b + 1) * 16] = dstib[b * 16:(b + 1) * 16]
            pltpu.async_copy(msgb, acc.at[dstqb], sadb, add=True)

            @pl.when(i < pairs - 1)
            def _():
                fire_lin(ja + 3, dstib, rvb, slb, srcib)
            return c

        lax.fori_loop(0, pairs, step, 0)
        pltpu.make_async_copy(msga, acc.at[dstqa], sada).wait()
        pltpu.make_async_copy(msgb, acc.at[dstqb], sadb).wait()

        toff = base + _SFULL * _MCH
        pltpu.sync_copy(dst.at[pl.ds(toff, _STAIL)], dstt)
        pltpu.sync_copy(rec.at[ci, pl.ds(toff, _STAIL)],
                        rva.at[pl.ds(0, _STAIL)])
        if has_wp:
            pltpu.sync_copy(src.at[pl.ds(toff, _STAIL)], srct)

            @pl.when(ci == 0)
            def _():
                pltpu.async_copy(h0t.at[srct], hva.at[pl.ds(0, _STAIL)],
                                 sha).wait()

            @pl.when(ci == 1)
            def _():
                pltpu.async_copy(h1t.at[srct], hva.at[pl.ds(0, _STAIL)],
                                 sha).wait()

        def erow_t(e, c):
            a1 = rva[e, 0:16]
            a2 = rva[e, 16:32]
            yrow = rva[e, 32:48]
            if has_wp:
                w0 = rva[e, 48:64]
                w1 = rva[e, 64:80]
            als = (None, a1, a1, a1, a2, a2, a2, a2, a2)
            for j in range(1, 9):
                m = als[j] * yrow[j]
                if has_wp:
                    w16 = w0 if j < 4 else w1
                    m = m + w16 * hva[e, (j - 1) * 16:j * 16]
                msga[e, (j - 1) * 16:j * 16] = m
            return c

        lax.fori_loop(0, _STAIL, erow_t, 0)
        pltpu.sync_copy(msga.at[pl.ds(0, _STAIL)], acc.at[dstt], add=True)

        plsc.subcore_barrier()
        pltpu.sync_copy(acc.at[pl.ds(t * _NROWS, _NROWS)],
                        agg_o.at[ci, pl.ds(t * _NROWS, _NROWS)])

    if has_wp:
        @functools.partial(pl.kernel, mesh=_mesh(), out_type=outs,
                           scratch_types=scr)
        def sk(rec, dst, src, h0t, h1t, agg_o,
               acc, dstia, dstib, dstqa, dstqb, dstt, rva, rvb, msga, msgb,
               sla, slb, sada, sadb, srcia, srcib, srct, hva, hvb, sha, shb):
            body(rec, dst, src, h0t, h1t, agg_o,
                 acc, dstia, dstib, dstqa, dstqb, dstt, rva, rvb, msga, msgb,
                 sla, slb, sada, sadb, srcia, srcib, srct, hva, hvb, sha, shb)
    else:
        @functools.partial(pl.kernel, mesh=_mesh(), out_type=outs,
                           scratch_types=scr)
        def sk(rec, dst, agg_o,
               acc, dstia, dstib, dstqa, dstqb, dstt, rva, rvb, msga, msgb,
               sla, slb, sada, sadb):
            body(rec, dst, None, None, None, agg_o,
                 acc, dstia, dstib, dstqa, dstqb, dstt, rva, rvb, msga, msgb,
                 sla, slb, sada, sadb)
    return sk


def _make_scatter0():
    outs = jax.ShapeDtypeStruct((2, _NPAD, 128), F32)
    scr = [
        pltpu.VMEM_SHARED((_NPAD, 128), F32),
        pltpu.VMEM((_SCH,), jnp.int32),
        pltpu.VMEM((_SCH,), jnp.int32),
        pltpu.VMEM((_PTAIL,), jnp.int32),
        pltpu.VMEM((_SCH, 128), F32),
        pltpu.VMEM((_SCH, 128), F32),
        pltpu.SemaphoreType.DMA,
        pltpu.SemaphoreType.DMA,
    ]

    @functools.partial(pl.kernel, mesh=_mesh(), out_type=outs,
                       scratch_types=scr)
    def sk(pay, dst, agg_o, acc, dstia, dstib, dstt, pva, pvb, sema, semb):
        ci = lax.axis_index("c")
        t = lax.axis_index("s")
        _acc_init(acc, pva, t)
        plsc.subcore_barrier()
        base = (t * _NC + ci) * _PPER
        pairs = _PFULL // 2

        def fire(j, di, pv, sem):
            off = base + j * _SCH
            pltpu.async_copy(dst.at[pl.ds(off, _SCH)], di, sem)
            pltpu.async_copy(pay.at[ci, pl.ds(off, _SCH)], pv, sem)

        def wfire(j, di, pv, sem):
            off = base + j * _SCH
            pltpu.make_async_copy(dst.at[pl.ds(off, _SCH)], di, sem).wait()
            pltpu.make_async_copy(pay.at[ci, pl.ds(off, _SCH)], pv,
                                  sem).wait()

        fire(0, dstia, pva, sema)
        fire(1, dstib, pvb, semb)

        def step(i, c):
            ja = 2 * i
            wfire(ja, dstia, pva, sema)
            pltpu.sync_copy(pva, acc.at[dstia], add=True)

            @pl.when(i < pairs - 1)
            def _():
                fire(ja + 2, dstia, pva, sema)

            wfire(ja + 1, dstib, pvb, semb)
            pltpu.sync_copy(pvb, acc.at[dstib], add=True)

            @pl.when(i < pairs - 1)
            def _():
                fire(ja + 3, dstib, pvb, semb)
            return c

        lax.fori_loop(0, pairs, step, 0)

        toff = base + _PFULL * _SCH
        pltpu.sync_copy(dst.at[pl.ds(toff, _PTAIL)], dstt)
        pltpu.sync_copy(pay.at[ci, pl.ds(toff, _PTAIL)],
                        pva.at[pl.ds(0, _PTAIL)])
        pltpu.sync_copy(pva.at[pl.ds(0, _PTAIL)], acc.at[dstt], add=True)

        plsc.subcore_barrier()
        pltpu.sync_copy(acc.at[pl.ds(t * _NROWS, _NROWS)],
                        agg_o.at[ci, pl.ds(t * _NROWS, _NROWS)])

    return sk


def _nodeP1_body(agg_ref, aggs_ref, lin_ref, o_ref, st_ref):
    i = pl.program_id(0)
    cnt = aggs_ref[0, :, 112:113] + aggs_ref[1, :, 112:113]
    inv = 1.0 / jnp.maximum(cnt, 1.0)
    os_ = []
    for k in range(9):
        if k == 0:
            xk = (aggs_ref[0, :, 80:112] + aggs_ref[1, :, 80:112]) * inv
        else:
            xk = jnp.concatenate(
                [agg_ref[0, :, (k - 1) * 16:k * 16],
                 agg_ref[1, :, (k - 1) * 16:k * 16]], axis=1) * inv
        os_.append(jnp.dot(xk, lin_ref[_LMAP[k]], preferred_element_type=F32))
    o_ref[...] = jnp.concatenate(os_, axis=1)
    s = os_[0]
    n1 = jnp.sqrt(os_[1] ** 2 + os_[2] ** 2 + os_[3] ** 2 + 1e-12)
    n2 = jnp.sqrt(os_[4] ** 2 + os_[5] ** 2 + os_[6] ** 2
                  + os_[7] ** 2 + os_[8] ** 2 + 1e-12)
    part = jnp.concatenate([
        jnp.sum(s, axis=0, keepdims=True),
        jnp.sum(s * s, axis=0, keepdims=True),
        jnp.sum(n1, axis=0, keepdims=True),
        jnp.sum(n2, axis=0, keepdims=True)], axis=0)

    @pl.when(i == 0)
    def _():
        st_ref[...] = part

    @pl.when(i != 0)
    def _():
        st_ref[...] = st_ref[...] + part


def _nodeP1(agg, aggs, lin):
    nb = _N // _NB
    full = lambda *s: pl.BlockSpec(s, lambda i: (0,) * len(s))
    return pl.pallas_call(
        _nodeP1_body,
        grid=(nb,),
        in_specs=[
            pl.BlockSpec((2, _NB, 128), lambda i: (0, i, 0)),
            pl.BlockSpec((2, _NB, 128), lambda i: (0, i, 0)),
            full(3, 32, 32),
        ],
        out_specs=[
            pl.BlockSpec((_NB, 288), lambda i: (i, 0)),
            full(4, 32),
        ],
        out_shape=[
            jax.ShapeDtypeStruct((_N, 288), F32),
            jax.ShapeDtypeStruct((4, 32), F32),
        ],
    )(agg, aggs, lin)


def _bn_apply(o_ref, st_ref):
    mu = st_ref[0:1, :] * (1.0 / _N)
    var = st_ref[1:2, :] * (1.0 / _N) - mu * mu
    sd = jnp.sqrt(jnp.maximum(var, 0.0))
    isd = 1.0 / (sd + 1e-5)
    in1 = 1.0 / (st_ref[2:3, :] * (1.0 / _N) + 1e-5)
    in2 = 1.0 / (st_ref[3:4, :] * (1.0 / _N) + 1e-5)
    aks = []
    for k in range(9):
        pk = o_ref[:, k * 32:(k + 1) * 32]
        if k == 0:
            aks.append((pk - mu) * isd)
        elif k < 4:
            aks.append(pk * in1)
        else:
            aks.append(pk * in2)
    return aks


def _nodeP2_body(o_ref, st_ref, pw_ref, plin_ref, o2_ref, st2_ref):
    i = pl.program_id(0)
    aks = _bn_apply(o_ref, st_ref)
    a0 = aks[0]
    f = pw_ref[0:1, :] + pw_ref[1:2, :] * a0 + pw_ref[2:3, :] * a0 * a0
    ps = [jnp.dot(f * aks[k], plin_ref[_LMAP[k]], preferred_element_type=F32)
          for k in range(9)]
    o2_ref[...] = jnp.concatenate(ps, axis=1)
    s = ps[0]
    n1 = jnp.sqrt(ps[1] ** 2 + ps[2] ** 2 + ps[3] ** 2 + 1e-12)
    n2 = jnp.sqrt(ps[4] ** 2 + ps[5] ** 2 + ps[6] ** 2
                  + ps[7] ** 2 + ps[8] ** 2 + 1e-12)
    part = jnp.concatenate([
        jnp.sum(s, axis=0, keepdims=True),
        jnp.sum(s * s, axis=0, keepdims=True),
        jnp.sum(n1, axis=0, keepdims=True),
        jnp.sum(n2, axis=0, keepdims=True)], axis=0)

    @pl.when(i == 0)
    def _():
        st2_ref[...] = part

    @pl.when(i != 0)
    def _():
        st2_ref[...] = st2_ref[...] + part


def _nodeP2(o1, st1, pw2d, plin):
    nb = _N // _NB
    full = lambda *s: pl.BlockSpec(s, lambda i: (0,) * len(s))
    return pl.pallas_call(
        _nodeP2_body,
        grid=(nb,),
        in_specs=[
            pl.BlockSpec((_NB, 288), lambda i: (i, 0)),
            full(4, 32), full(3, 32), full(3, 32, 32),
        ],
        out_specs=[
            pl.BlockSpec((_NB, 288), lambda i: (i, 0)),
            full(4, 32),
        ],
        out_shape=[
            jax.ShapeDtypeStruct((_N, 288), F32),
            jax.ShapeDtypeStruct((4, 32), F32),
        ],
    )(o1, st1, pw2d, plin)


def _nodeP3_body(o2_ref, st_ref, h0_ref, hs2_ref, hsc_ref):
    bks = _bn_apply(o2_ref, st_ref)
    b0 = bks[0] + h0_ref[:, 0:32]
    hs2_ref[...] = jnp.concatenate(
        [b0, jnp.zeros((b0.shape[0], 96), F32)], axis=1)
    for ci in range(2):
        hsc_ref[ci] = jnp.concatenate(
            [bks[k][:, ci * 16:(ci + 1) * 16] for k in range(1, 9)], axis=1)


def _nodeP3(o2, st2, tab):
    nb = _N // _NB
    full = lambda *s: pl.BlockSpec(s, lambda i: (0,) * len(s))
    return pl.pallas_call(
        _nodeP3_body,
        grid=(nb,),
        in_specs=[
            pl.BlockSpec((_NB, 288), lambda i: (i, 0)),
            full(4, 32),
            pl.BlockSpec((_NB, 128), lambda i: (i, 0)),
        ],
        out_specs=[
            pl.BlockSpec((_NB, 128), lambda i: (i, 0)),
            pl.BlockSpec((2, _NB, 128), lambda i: (0, i, 0)),
        ],
        out_shape=[
            jax.ShapeDtypeStruct((_N, 128), F32),
            jax.ShapeDtypeStruct((2, _N, 128), F32),
        ],
    )(o2, st2, tab)


def _nodeQ_body(o_ref, st_ref, pw_ref, plin_ref, q_ref, st2_ref):
    i = pl.program_id(0)
    aks = _bn_apply(o_ref, st_ref)
    a0 = aks[0]
    f = pw_ref[0:1, :] + pw_ref[1:2, :] * a0 + pw_ref[2:3, :] * a0 * a0
    q = jnp.dot(f * a0, plin_ref[0], preferred_element_type=F32)
    q_ref[...] = q
    part = jnp.concatenate([
        jnp.sum(q, axis=0, keepdims=True),
        jnp.sum(q * q, axis=0, keepdims=True)], axis=0)

    @pl.when(i == 0)
    def _():
        st2_ref[...] = part

    @pl.when(i != 0)
    def _():
        st2_ref[...] = st2_ref[...] + part


def _nodeQ(o1, st1, pw2d, plin):
    nb = _N // _NB
    full = lambda *s: pl.BlockSpec(s, lambda i: (0,) * len(s))
    return pl.pallas_call(
        _nodeQ_body,
        grid=(nb,),
        in_specs=[
            pl.BlockSpec((_NB, 288), lambda i: (i, 0)),
            full(4, 32), full(3, 32), full(3, 32, 32),
        ],
        out_specs=[
            pl.BlockSpec((_NB, 32), lambda i: (i, 0)),
            full(2, 32),
        ],
        out_shape=[
            jax.ShapeDtypeStruct((_N, 32), F32),
            jax.ShapeDtypeStruct((2, 32), F32),
        ],
    )(o1, st1, pw2d, plin)


def _nodeF_body(q_ref, st_ref, b_ref, node_ref, graph_ref):
    i = pl.program_id(0)
    mu = st_ref[0:1, :] * (1.0 / _N)
    var = st_ref[1:2, :] * (1.0 / _N) - mu * mu
    sd = jnp.sqrt(jnp.maximum(var, 0.0))
    node = (q_ref[...] - mu) / (sd + 1e-5)
    node_ref[...] = node
    bt = b_ref[0, 0, :]
    oht = (lax.broadcasted_iota(jnp.int32, (64, _NB), 0)
           == bt[None, :]).astype(F32)
    g = jnp.dot(oht, node, preferred_element_type=F32)

    @pl.when(i == 0)
    def _():
        graph_ref[...] = g

    @pl.when(i != 0)
    def _():
        graph_ref[...] = graph_ref[...] + g


def _nodeF(q, st3, batch3):
    nb = _N // _NB
    full = lambda *s: pl.BlockSpec(s, lambda i: (0,) * len(s))
    return pl.pallas_call(
        _nodeF_body,
        grid=(nb,),
        in_specs=[
            pl.BlockSpec((_NB, 32), lambda i: (i, 0)),
            full(2, 32),
            pl.BlockSpec((1, 1, _NB), lambda i: (i, 0, 0)),
        ],
        out_specs=[
            pl.BlockSpec((_NB, 32), lambda i: (i, 0)),
            full(64, 32),
        ],
        out_shape=[
            jax.ShapeDtypeStruct((_N, 32), F32),
            jax.ShapeDtypeStruct((64, 32), F32),
        ],
    )(q, st3, batch3)


def _perm_ws():
    return np.array([3 * c + l for l in range(3) for c in range(32)])


def _perm_wp():
    return np.array([2 * c + m for m in range(2) for c in range(32)])


_gather128 = _make_gather()
_scatter1 = _make_scatter(False)
_scatter2 = _make_scatter(True)
_scatter0 = _make_scatter0()


def kernel(x, pos, edge_index, batch, W_emb, b_emb, mW1_0, mb1_0, mW2_0,
           mb2_0, lin_0, pw_0, plin_0, mW1_1, mb1_1, mW2_1, mb2_1, lin_1,
           pw_1, plin_1):
    e0 = edge_index[0].astype(jnp.int32)
    e1 = edge_index[1].astype(jnp.int32)
    src_all = jnp.concatenate([e0, e1])
    dst_all = jnp.concatenate([e1, e0])

    pws = _perm_ws()
    pwp = _perm_wp()
    w1e_0, w1a_0, w1b_0 = mW1_0[0:8], mW1_0[8:40], mW1_0[40:72]
    w1e_1, w1a_1, w1b_1 = mW1_1[0:8], mW1_1[8:40], mW1_1[40:72]
    w2a_0 = mW2_0[:, pws]
    b2a_0 = mb2_0[pws][None, :]
    w2a_1 = mW2_1[:, :96][:, pws]
    b2a_1 = mb2_1[:96][pws][None, :]
    w2p_1 = mW2_1[:, 96:160][:, pwp]
    b2p_1 = mb2_1[96:160][pwp][None, :]

    tab = _embed(x, pos, W_emb, b_emb[None, :])
    g0, g1 = _gather128(tab, e0, e1)
    rec1, yx, ef = _edge1(g0, g1, w1e_0, w1a_0, w1b_0, mb1_0[None, :],
                          w2a_0, b2a_0)
    rec1 = rec1.reshape(2, _E2, 128)
    agg1 = _scatter1(rec1, dst_all)
    aggs1 = _scatter0(rec1, dst_all)
    o1, st1 = _nodeP1(agg1, aggs1, lin_0)
    o2, st2 = _nodeP2(o1, st1, pw_0, plin_0)
    hs2, h2sc = _nodeP3(o2, st2, tab)

    g20, g21 = _gather128(hs2, e0, e1)
    rec2 = _edge2(g20, g21, ef, yx, w1e_1, w1a_1, w1b_1,
                  mb1_1[None, :], w2a_1, b2a_1, w2p_1, b2p_1)
    rec2 = rec2.reshape(2, _E2, 128)
    agg2 = _scatter2(rec2, dst_all, src_all, h2sc[0], h2sc[1])
    aggs2 = _scatter0(rec2, dst_all)
    o1b, st1b = _nodeP1(agg2, aggs2, lin_1)
    q, st3 = _nodeQ(o1b, st1b, pw_1, plin_1)
    node, graph = _nodeF(q, st3,
                         batch.astype(jnp.int32).reshape(_N // _NB, 1, _NB))
    return (node, graph)


# speedup vs baseline: 1.0638x; 1.0012x over previous
import functools

import jax
import jax.numpy as jnp
import numpy as np
from jax import lax
from jax.experimental import pallas as pl
from jax.experimental.pallas import tpu as pltpu
from jax.experimental.pallas import tpu_sc as plsc

F32 = jnp.float32
_N = 10000
_E = 160000
_E2 = 2 * _E
_EMB = 32
_R_MAX = 10.0
_NB = 1000
_EB = 1000
_LMAP = [0, 1, 1, 1, 2, 2, 2, 2, 2]
_S3 = 3.0 ** 0.5
_S15 = 15.0 ** 0.5
_S5 = 5.0 ** 0.5

_NC, _NS = 2, 16
_NW = _NC * _NS
_GPER = _E // _NW
_SPER = _E2 // _NS
_SCH = 128
_GFULL = _GPER // _SCH
_GTAIL = _GPER - _GFULL * _SCH
_MCH = 64
_STAIL = 32
_SFULL = (_SPER - _STAIL) // _MCH
_PPER = _E2 // _NW
_PFULL = _PPER // _SCH
_PTAIL = _PPER - _PFULL * _SCH
_NPAD = 10112
_NROWS = _NPAD // _NS


def _mesh():
    return plsc.VectorSubcoreMesh(core_axis_name="c", subcore_axis_name="s")


def _embed_body(x_ref, pos_ref, w_ref, b_ref, tab_ref):
    h = jnp.dot(x_ref[...], w_ref[...], preferred_element_type=F32) + b_ref[...]
    z = jnp.zeros((x_ref.shape[0], 93), F32)
    tab_ref[...] = jnp.concatenate([h, pos_ref[...], z], axis=1)


def _embed(x, pos, w, b2d):
    nb = _N // _NB
    return pl.pallas_call(
        _embed_body,
        grid=(nb,),
        in_specs=[
            pl.BlockSpec((_NB, 128), lambda i: (i, 0)),
            pl.BlockSpec((_NB, 3), lambda i: (i, 0)),
            pl.BlockSpec((128, _EMB), lambda i: (0, 0)),
            pl.BlockSpec((1, _EMB), lambda i: (0, 0)),
        ],
        out_specs=pl.BlockSpec((_NB, 128), lambda i: (i, 0)),
        out_shape=jax.ShapeDtypeStruct((_N, 128), F32),
    )(x, pos, w, b2d)


def _make_gather():
    @functools.partial(
        pl.kernel,
        mesh=_mesh(),
        out_type=[jax.ShapeDtypeStruct((_E, 128), F32),
                  jax.ShapeDtypeStruct((_E, 128), F32)],
        scratch_types=[
            pltpu.VMEM((_GPER,), jnp.int32),
            pltpu.VMEM((_SCH, 128), F32),
            pltpu.VMEM((_SCH, 128), F32),
            pltpu.SemaphoreType.DMA,
            pltpu.SemaphoreType.DMA,
        ],
    )
    def gk(tab, i0, i1, o0, o1, idxv, bufa, bufb, sema, semb):
        w = lax.axis_index("s") * _NC + lax.axis_index("c")
        base = w * _GPER

        for ih, oh in ((i0, o0), (i1, o1)):
            pltpu.sync_copy(ih.at[pl.ds(base, _GPER)], idxv)

            def fire(j, buf, sem):
                return pltpu.async_copy(
                    tab.at[idxv.at[pl.ds(j * _SCH, _SCH)]], buf, sem)

            def wfire(j, buf, sem):
                pltpu.make_async_copy(
                    tab.at[idxv.at[pl.ds(j * _SCH, _SCH)]], buf, sem).wait()

            fire(0, bufa, sema)
            fire(1, bufb, semb)

            def step(i, c, oh=oh):
                ja = 2 * i
                wfire(ja, bufa, sema)
                pltpu.sync_copy(bufa, oh.at[pl.ds(base + ja * _SCH, _SCH)])
                fire(ja + 2, bufa, sema)
                wfire(ja + 1, bufb, semb)
                pltpu.sync_copy(bufb, oh.at[pl.ds(base + (ja + 1) * _SCH, _SCH)])

                @pl.when(i < (_GFULL - 1) // 2 - 1)
                def _():
                    fire(ja + 3, bufb, semb)
                return c

            lax.fori_loop(0, (_GFULL - 1) // 2, step, 0)
            wfire(_GFULL - 1, bufa, sema)
            pltpu.sync_copy(bufa,
                            oh.at[pl.ds(base + (_GFULL - 1) * _SCH, _SCH)])
            pltpu.async_copy(
                tab.at[idxv.at[pl.ds(_GFULL * _SCH, _GTAIL)]],
                bufa.at[pl.ds(0, _GTAIL)], sema).wait()
            pltpu.sync_copy(bufa.at[pl.ds(0, _GTAIL)],
                            oh.at[pl.ds(base + _GFULL * _SCH, _GTAIL)])

    return gk


def _mlp_tail(ef, hs0, hs1, w1e_ref, w1a_ref, w1b_ref, b1_ref,
              w2a_ref, b2a_ref):
    base = jnp.dot(ef, w1e_ref[...], preferred_element_type=F32) + b1_ref[...]
    p0a = jnp.dot(hs0, w1a_ref[...], preferred_element_type=F32)
    p0b = jnp.dot(hs0, w1b_ref[...], preferred_element_type=F32)
    p1a = jnp.dot(hs1, w1a_ref[...], preferred_element_type=F32)
    p1b = jnp.dot(hs1, w1b_ref[...], preferred_element_type=F32)
    hf = jax.nn.relu(base + p0a + p1b)
    hr = jax.nn.relu(base + p1a + p0b)
    taf = jnp.dot(hf, w2a_ref[...], preferred_element_type=F32) + b2a_ref[...]
    tar = jnp.dot(hr, w2a_ref[...], preferred_element_type=F32) + b2a_ref[...]
    stf = jnp.concatenate([hs0, hs0, hs0], axis=1)
    str_ = jnp.concatenate([hs1, hs1, hs1], axis=1)
    return (hf, hr), (taf * stf, tar * str_)


def _pack_rec(rec_ref, a, wp, yf16, par):
    ys = (yf16, yf16 * par)
    one = jnp.ones((_EB, 16), F32)
    z32 = jnp.zeros((_EB, 32), F32)
    for ci in range(2):
        for d in range(2):
            ad = a[d]
            parts = [ad[:, 32 + ci * 16:48 + ci * 16],
                     ad[:, 64 + ci * 16:80 + ci * 16],
                     ys[d]]
            if wp is not None:
                wd = wp[d]
                parts += [wd[:, ci * 16:16 + ci * 16],
                          wd[:, 32 + ci * 16:48 + ci * 16]]
            else:
                parts.append(z32)
            parts += [ad[:, 0:32], one]
            rec_ref[ci, d] = jnp.concatenate(parts, axis=1)


def _edge1_body(g0_ref, g1_ref, w1e_ref, w1a_ref, w1b_ref, b1_ref,
                w2a_ref, b2a_ref, rec_ref, y_ref, ef_ref):
    g0 = g0_ref[...]
    g1 = g1_ref[...]
    hs0 = g0[:, 0:32]
    hs1 = g1[:, 0:32]
    dd = g0[:, 32:35] - g1[:, 32:35]
    r = jnp.sqrt(jnp.sum(dd * dd, axis=1, keepdims=True))
    vh = dd / (r + 1e-9)
    vx, vy, vz = vh[:, 0:1], vh[:, 1:2], vh[:, 2:3]
    zer7 = jnp.zeros((_EB, 7), F32)
    yf = jnp.concatenate([
        jnp.ones_like(vx), _S3 * vy, _S3 * vz, _S3 * vx,
        _S15 * vx * vy, _S15 * vy * vz, (_S5 / 2.0) * (3.0 * vz * vz - 1.0),
        _S15 * vx * vz, (_S15 / 2.0) * (vx * vx - vy * vy), zer7], axis=1)
    nrow = (lax.broadcasted_iota(jnp.int32, (1, 8), 1) + 1).astype(F32)
    bes = (2.0 / _R_MAX) ** 0.5 * jnp.sin(nrow * (np.pi / _R_MAX) * r) / (r + 1e-9)
    xr = r * (1.0 / _R_MAX)
    x2 = xr * xr
    x5 = x2 * x2 * xr
    env = 1.0 - 21.0 * x5 + 35.0 * x5 * xr - 15.0 * x5 * x2
    env = jnp.where(xr < 1.0, env, 0.0)
    ef = bes * env
    _, a = _mlp_tail(ef, hs0, hs1, w1e_ref, w1a_ref, w1b_ref, b1_ref,
                     w2a_ref, b2a_ref)
    col = lax.broadcasted_iota(jnp.int32, (1, 16), 1)
    par = jnp.where((col >= 1) & (col <= 3), -1.0, 1.0).astype(F32)
    _pack_rec(rec_ref, a, None, yf, par)
    y_ref[...] = yf
    ef_ref[...] = ef


def _edge1(g0, g1, w1e, w1a, w1b, b1, w2a, b2a):
    nb = _E // _EB
    full = lambda *s: pl.BlockSpec(s, lambda i: (0,) * len(s))
    return pl.pallas_call(
        _edge1_body,
        grid=(nb,),
        in_specs=[
            pl.BlockSpec((_EB, 128), lambda i: (i, 0)),
            pl.BlockSpec((_EB, 128), lambda i: (i, 0)),
            full(8, 256), full(32, 256), full(32, 256), full(1, 256),
            full(256, 96), full(1, 96),
        ],
        out_specs=[
            pl.BlockSpec((2, 2, _EB, 128), lambda i: (0, 0, i, 0)),
            pl.BlockSpec((_EB, 16), lambda i: (i, 0)),
            pl.BlockSpec((_EB, 8), lambda i: (i, 0)),
        ],
        out_shape=[
            jax.ShapeDtypeStruct((2, 2, _E, 128), F32),
            jax.ShapeDtypeStruct((_E, 16), F32),
            jax.ShapeDtypeStruct((_E, 8), F32),
        ],
    )(g0, g1, w1e, w1a, w1b, b1, w2a, b2a)


def _edge2_body(g0_ref, g1_ref, ef_ref, y_ref, w1e_ref, w1a_ref, w1b_ref,
                b1_ref, w2a_ref, b2a_ref, w2p_ref, b2p_ref, rec_ref):
    hs0 = g0_ref[:, 0:32]
    hs1 = g1_ref[:, 0:32]
    ef = ef_ref[...]
    yf = y_ref[...]
    hid, a = _mlp_tail(ef, hs0, hs1, w1e_ref, w1a_ref, w1b_ref, b1_ref,
                       w2a_ref, b2a_ref)
    tp = tuple(
        jnp.dot(h, w2p_ref[...], preferred_element_type=F32) + b2p_ref[...]
        for h in hid)
    col = lax.broadcasted_iota(jnp.int32, (1, 16), 1)
    par = jnp.where((col >= 1) & (col <= 3), -1.0, 1.0).astype(F32)
    _pack_rec(rec_ref, a, tp, yf, par)


def _edge2(g0, g1, ef, yx, w1e, w1a, w1b, b1, w2a, b2a, w2p, b2p):
    nb = _E // _EB
    full = lambda *s: pl.BlockSpec(s, lambda i: (0,) * len(s))
    return pl.pallas_call(
        _edge2_body,
        grid=(nb,),
        in_specs=[
            pl.BlockSpec((_EB, 128), lambda i: (i, 0)),
            pl.BlockSpec((_EB, 128), lambda i: (i, 0)),
            pl.BlockSpec((_EB, 8), lambda i: (i, 0)),
            pl.BlockSpec((_EB, 16), lambda i: (i, 0)),
            full(8, 256), full(32, 256), full(32, 256), full(1, 256),
            full(256, 96), full(1, 96),
            full(256, 64), full(1, 64),
        ],
        out_specs=pl.BlockSpec((2, 2, _EB, 128), lambda i: (0, 0, i, 0)),
        out_shape=jax.ShapeDtypeStruct((2, 2, _E, 128), F32),
    )(g0, g1, ef, yx, w1e, w1a, w1b, b1, w2a, b2a, w2p, b2p)


def _zero_fill(buf, rows, cols16):
    def z(e, c):
        for q in range(cols16):
            buf[e, q * 16:(q + 1) * 16] = jnp.zeros((16,), F32)
        return c
    lax.fori_loop(0, rows, z, 0)


def _acc_init(acc, zsrc, t):
    rows = zsrc.shape[0]
    _zero_fill(zsrc, rows, 8)
    nfull = _NROWS // rows
    for q in range(nfull):
        pltpu.sync_copy(zsrc, acc.at[pl.ds(t * _NROWS + q * rows, rows)])
    rem = _NROWS - nfull * rows
    if rem:
        pltpu.sync_copy(zsrc.at[pl.ds(0, rem)],
                        acc.at[pl.ds(t * _NROWS + nfull * rows, rem)])


def _make_scatter(has_wp):
    outs = jax.ShapeDtypeStruct((2, _NPAD, 128), F32)
    scr = [
        pltpu.VMEM_SHARED((_NPAD, 128), F32),
        pltpu.VMEM((_MCH,), jnp.int32),
        pltpu.VMEM((_MCH,), jnp.int32),
        pltpu.VMEM((_MCH,), jnp.int32),
        pltpu.VMEM((_MCH,), jnp.int32),
        pltpu.VMEM((_STAIL,), jnp.int32),
        pltpu.VMEM((_MCH, 128), F32),
        pltpu.VMEM((_MCH, 128), F32),
        pltpu.VMEM((_MCH, 128), F32),
        pltpu.VMEM((_MCH, 128), F32),
        pltpu.SemaphoreType.DMA,
        pltpu.SemaphoreType.DMA,
        pltpu.SemaphoreType.DMA,
        pltpu.SemaphoreType.DMA,
    ]
    if has_wp:
        scr += [
            pltpu.VMEM((_MCH,), jnp.int32),
            pltpu.VMEM((_MCH,), jnp.int32),
            pltpu.VMEM((_STAIL,), jnp.int32),
            pltpu.VMEM((_MCH, 128), F32),
            pltpu.VMEM((_MCH, 128), F32),
            pltpu.SemaphoreType.DMA,
            pltpu.SemaphoreType.DMA,
        ]

    def body(rec, dst, src, h0t, h1t, agg_o,
             acc, dstia, dstib, dstqa, dstqb, dstt, rva, rvb, msga, msgb,
             sla, slb, sada, sadb,
             srcia=None, srcib=None, srct=None, hva=None, hvb=None,
             sha=None, shb=None):
        ci = lax.axis_index("c")
        t = lax.axis_index("s")
        _acc_init(acc, msga, t)
        plsc.subcore_barrier()
        base = t * _SPER
        pairs = _SFULL // 2

        def fire_lin(j, di, rvx, sem, si):
            off = base + j * _MCH
            pltpu.async_copy(dst.at[pl.ds(off, _MCH)], di, sem)
            pltpu.async_copy(rec.at[ci, pl.ds(off, _MCH)], rvx, sem)
            if has_wp:
                pltpu.async_copy(src.at[pl.ds(off, _MCH)], si, sem)

        def wait_lin(j, di, rvx, sem, si):
            off = base + j * _MCH
            pltpu.make_async_copy(dst.at[pl.ds(off, _MCH)], di, sem).wait()
            pltpu.make_async_copy(rec.at[ci, pl.ds(off, _MCH)], rvx,
                                  sem).wait()
            if has_wp:
                pltpu.make_async_copy(src.at[pl.ds(off, _MCH)], si,
                                      sem).wait()

        def fire_hg(si, hvx, sem):
            @pl.when(ci == 0)
            def _():
                pltpu.async_copy(h0t.at[si], hvx, sem)

            @pl.when(ci == 1)
            def _():
                pltpu.async_copy(h1t.at[si], hvx, sem)

        def wait_hg(si, hvx, sem):
            @pl.when(ci == 0)
            def _():
                pltpu.make_async_copy(h0t.at[si], hvx, sem).wait()

            @pl.when(ci == 1)
            def _():
                pltpu.make_async_copy(h1t.at[si], hvx, sem).wait()

        def compute(k, rvx, hvx, msgx):
            @functools.partial(plsc.parallel_loop, 0, k, unroll=4)
            def erow(e):
                a1 = rvx[e, 0:16]
                a2 = rvx[e, 16:32]
                yrow = rvx[e, 32:48]
                if has_wp:
                    w0 = rvx[e, 48:64]
                    w1 = rvx[e, 64:80]
                als = (None, a1, a1, a1, a2, a2, a2, a2, a2)
                for j in range(1, 9):
                    m = als[j] * yrow[j]
                    if has_wp:
                        w16 = w0 if j < 4 else w1
                        m = m + w16 * hvx[e, (j - 1) * 16:j * 16]
                    msgx[e, (j - 1) * 16:j * 16] = m

        fire_lin(0, dstia, rva, sla, srcia)
        wait_lin(0, dstia, rva, sla, srcia)
        if has_wp:
            fire_hg(srcia, hva, sha)
        fire_lin(1, dstib, rvb, slb, srcib)

        def step(i, c):
            ja = 2 * i
            wait_lin(ja + 1, dstib, rvb, slb, srcib)
            if has_wp:
                fire_hg(srcib, hvb, shb)
                wait_hg(srcia, hva, sha)

            @pl.when(i > 0)
            def _():
                pltpu.make_async_copy(msga, acc.at[dstqa], sada).wait()

            compute(_MCH, rva, hva, msga)
            for b in range(_MCH // 16):
                dstqa[b * 16:(b + 1) * 16] = dstia[b * 16:(b + 1) * 16]
            pltpu.async_copy(msga, acc.at[dstqa], sada, add=True)

            @pl.when(i < pairs - 1)
            def _():
                fire_lin(ja + 2, dstia, rva, sla, srcia)
                wait_lin(ja + 2, dstia, rva, sla, srcia)
                if has_wp:
                    fire_hg(srcia, hva, sha)

            if has_wp:
                wait_hg(srcib, hvb, shb)

            @pl.when(i > 0)
            def _():
                pltpu.make_async_copy(msgb, acc.at[dstqb], sadb).wait()

            compute(_MCH, rvb, hvb, msgb)
            for b in range(_MCH // 16):
                dstqb[b * 16:(b + 1) * 16] = dstib[b * 16:(b + 1) * 16]
            pltpu.async_copy(msgb, acc.at[dstqb], sadb, add=True)

            @pl.when(i < pairs - 1)
            def _():
                fire_lin(ja + 3, dstib, rvb, slb, srcib)
            return c

        lax.fori_loop(0, pairs, step, 0)
        pltpu.make_async_copy(msga, acc.at[dstqa], sada).wait()
        pltpu.make_async_copy(msgb, acc.at[dstqb], sadb).wait()

        toff = base + _SFULL * _MCH
        pltpu.sync_copy(dst.at[pl.ds(toff, _STAIL)], dstt)
        pltpu.sync_copy(rec.at[ci, pl.ds(toff, _STAIL)],
                        rva.at[pl.ds(0, _STAIL)])
        if has_wp:
            pltpu.sync_copy(src.at[pl.ds(toff, _STAIL)], srct)

            @pl.when(ci == 0)
            def _():
                pltpu.async_copy(h0t.at[srct], hva.at[pl.ds(0, _STAIL)],
                                 sha).wait()

            @pl.when(ci == 1)
            def _():
                pltpu.async_copy(h1t.at[srct], hva.at[pl.ds(0, _STAIL)],
                                 sha).wait()

        def erow_t(e, c):
            a1 = rva[e, 0:16]
            a2 = rva[e, 16:32]
            yrow = rva[e, 32:48]
            if has_wp:
                w0 = rva[e, 48:64]
                w1 = rva[e, 64:80]
            als = (None, a1, a1, a1, a2, a2, a2, a2, a2)
            for j in range(1, 9):
                m = als[j] * yrow[j]
                if has_wp:
                    w16 = w0 if j < 4 else w1
                    m = m + w16 * hva[e, (j - 1) * 16:j * 16]
                msga[e, (j - 1) * 16:j * 16] = m
            return c

        lax.fori_loop(0, _STAIL, erow_t, 0)
        pltpu.sync_copy(msga.at[pl.ds(0, _STAIL)], acc.at[dstt], add=True)

        plsc.subcore_barrier()
        pltpu.sync_copy(acc.at[pl.ds(t * _NROWS, _NROWS)],
                        agg_o.at[ci, pl.ds(t * _NROWS, _NROWS)])

    if has_wp:
        @functools.partial(pl.kernel, mesh=_mesh(), out_type=outs,
                           scratch_types=scr)
        def sk(rec, dst, src, h0t, h1t, agg_o,
               acc, dstia, dstib, dstqa, dstqb, dstt, rva, rvb, msga, msgb,
               sla, slb, sada, sadb, srcia, srcib, srct, hva, hvb, sha, shb):
            body(rec, dst, src, h0t, h1t, agg_o,
                 acc, dstia, dstib, dstqa, dstqb, dstt, rva, rvb, msga, msgb,
                 sla, slb, sada, sadb, srcia, srcib, srct, hva, hvb, sha, shb)
    else:
        @functools.partial(pl.kernel, mesh=_mesh(), out_type=outs,
                           scratch_types=scr)
        def sk(rec, dst, agg_o,
               acc, dstia, dstib, dstqa, dstqb, dstt, rva, rvb, msga, msgb,
               sla, slb, sada, sadb):
            body(rec, dst, None, None, None, agg_o,
                 acc, dstia, dstib, dstqa, dstqb, dstt, rva, rvb, msga, msgb,
                 sla, slb, sada, sadb)
    return sk


def _make_scatter0():
    outs = jax.ShapeDtypeStruct((2, _NPAD, 128), F32)
    scr = [
        pltpu.VMEM_SHARED((_NPAD, 128), F32),
        pltpu.VMEM((_SCH,), jnp.int32),
        pltpu.VMEM((_SCH,), jnp.int32),
        pltpu.VMEM((_PTAIL,), jnp.int32),
        pltpu.VMEM((_SCH, 128), F32),
        pltpu.VMEM((_SCH, 128), F32),
        pltpu.SemaphoreType.DMA,
        pltpu.SemaphoreType.DMA,
    ]

    @functools.partial(pl.kernel, mesh=_mesh(), out_type=outs,
                       scratch_types=scr)
    def sk(pay, dst, agg_o, acc, dstia, dstib, dstt, pva, pvb, sema, semb):
        ci = lax.axis_index("c")
        t = lax.axis_index("s")
        _acc_init(acc, pva, t)
        plsc.subcore_barrier()
        base = (t * _NC + ci) * _PPER
        pairs = _PFULL // 2

        def fire(j, di, pv, sem):
            off = base + j * _SCH
            pltpu.async_copy(dst.at[pl.ds(off, _SCH)], di, sem)
            pltpu.async_copy(pay.at[ci, pl.ds(off, _SCH)], pv, sem)

        def wfire(j, di, pv, sem):
            off = base + j * _SCH
            pltpu.make_async_copy(dst.at[pl.ds(off, _SCH)], di, sem).wait()
            pltpu.make_async_copy(pay.at[ci, pl.ds(off, _SCH)], pv,
                                  sem).wait()

        fire(0, dstia, pva, sema)
        fire(1, dstib, pvb, semb)

        def step(i, c):
            ja = 2 * i
            wfire(ja, dstia, pva, sema)
            pltpu.sync_copy(pva, acc.at[dstia], add=True)

            @pl.when(i < pairs - 1)
            def _():
                fire(ja + 2, dstia, pva, sema)

            wfire(ja + 1, dstib, pvb, semb)
            pltpu.sync_copy(pvb, acc.at[dstib], add=True)

            @pl.when(i < pairs - 1)
            def _():
                fire(ja + 3, dstib, pvb, semb)
            return c

        lax.fori_loop(0, pairs, step, 0)

        toff = base + _PFULL * _SCH
        pltpu.sync_copy(dst.at[pl.ds(toff, _PTAIL)], dstt)
        pltpu.sync_copy(pay.at[ci, pl.ds(toff, _PTAIL)],
                        pva.at[pl.ds(0, _PTAIL)])
        pltpu.sync_copy(pva.at[pl.ds(0, _PTAIL)], acc.at[dstt], add=True)

        plsc.subcore_barrier()
        pltpu.sync_copy(acc.at[pl.ds(t * _NROWS, _NROWS)],
                        agg_o.at[ci, pl.ds(t * _NROWS, _NROWS)])

    return sk


def _nodeP1_body(agg_ref, aggs_ref, lin_ref, o_ref, st_ref):
    i = pl.program_id(0)
    cnt = aggs_ref[0, :, 112:113] + aggs_ref[1, :, 112:113]
    inv = 1.0 / jnp.maximum(cnt, 1.0)
    os_ = []
    for k in range(9):
        if k == 0:
            xk = (aggs_ref[0, :, 80:112] + aggs_ref[1, :, 80:112]) * inv
        else:
            xk = jnp.concatenate(
                [agg_ref[0, :, (k - 1) * 16:k * 16],
                 agg_ref[1, :, (k - 1) * 16:k * 16]], axis=1) * inv
        os_.append(jnp.dot(xk, lin_ref[_LMAP[k]], preferred_element_type=F32))
    o_ref[...] = jnp.concatenate(os_, axis=1)
    s = os_[0]
    n1 = jnp.sqrt(os_[1] ** 2 + os_[2] ** 2 + os_[3] ** 2 + 1e-12)
    n2 = jnp.sqrt(os_[4] ** 2 + os_[5] ** 2 + os_[6] ** 2
                  + os_[7] ** 2 + os_[8] ** 2 + 1e-12)
    part = jnp.concatenate([
        jnp.sum(s, axis=0, keepdims=True),
        jnp.sum(s * s, axis=0, keepdims=True),
        jnp.sum(n1, axis=0, keepdims=True),
        jnp.sum(n2, axis=0, keepdims=True)], axis=0)

    @pl.when(i == 0)
    def _():
        st_ref[...] = part

    @pl.when(i != 0)
    def _():
        st_ref[...] = st_ref[...] + part


def _nodeP1(agg, aggs, lin):
    nb = _N // _NB
    full = lambda *s: pl.BlockSpec(s, lambda i: (0,) * len(s))
    return pl.pallas_call(
        _nodeP1_body,
        grid=(nb,),
        in_specs=[
            pl.BlockSpec((2, _NB, 128), lambda i: (0, i, 0)),
            pl.BlockSpec((2, _NB, 128), lambda i: (0, i, 0)),
            full(3, 32, 32),
        ],
        out_specs=[
            pl.BlockSpec((_NB, 288), lambda i: (i, 0)),
            full(4, 32),
        ],
        out_shape=[
            jax.ShapeDtypeStruct((_N, 288), F32),
            jax.ShapeDtypeStruct((4, 32), F32),
        ],
    )(agg, aggs, lin)


def _bn_apply(o_ref, st_ref):
    mu = st_ref[0:1, :] * (1.0 / _N)
    var = st_ref[1:2, :] * (1.0 / _N) - mu * mu
    sd = jnp.sqrt(jnp.maximum(var, 0.0))
    isd = 1.0 / (sd + 1e-5)
    in1 = 1.0 / (st_ref[2:3, :] * (1.0 / _N) + 1e-5)
    in2 = 1.0 / (st_ref[3:4, :] * (1.0 / _N) + 1e-5)
    aks = []
    for k in range(9):
        pk = o_ref[:, k * 32:(k + 1) * 32]
        if k == 0:
            aks.append((pk - mu) * isd)
        elif k < 4:
            aks.append(pk * in1)
        else:
            aks.append(pk * in2)
    return aks


def _nodeP2_body(o_ref, st_ref, pw_ref, plin_ref, o2_ref, st2_ref):
    i = pl.program_id(0)
    aks = _bn_apply(o_ref, st_ref)
    a0 = aks[0]
    f = pw_ref[0:1, :] + pw_ref[1:2, :] * a0 + pw_ref[2:3, :] * a0 * a0
    ps = [jnp.dot(f * aks[k], plin_ref[_LMAP[k]], preferred_element_type=F32)
          for k in range(9)]
    o2_ref[...] = jnp.concatenate(ps, axis=1)
    s = ps[0]
    n1 = jnp.sqrt(ps[1] ** 2 + ps[2] ** 2 + ps[3] ** 2 + 1e-12)
    n2 = jnp.sqrt(ps[4] ** 2 + ps[5] ** 2 + ps[6] ** 2
                  + ps[7] ** 2 + ps[8] ** 2 + 1e-12)
    part = jnp.concatenate([
        jnp.sum(s, axis=0, keepdims=True),
        jnp.sum(s * s, axis=0, keepdims=True),
        jnp.sum(n1, axis=0, keepdims=True),
        jnp.sum(n2, axis=0, keepdims=True)], axis=0)

    @pl.when(i == 0)
    def _():
        st2_ref[...] = part

    @pl.when(i != 0)
    def _():
        st2_ref[...] = st2_ref[...] + part


def _nodeP2(o1, st1, pw2d, plin):
    nb = _N // _NB
    full = lambda *s: pl.BlockSpec(s, lambda i: (0,) * len(s))
    return pl.pallas_call(
        _nodeP2_body,
        grid=(nb,),
        in_specs=[
            pl.BlockSpec((_NB, 288), lambda i: (i, 0)),
            full(4, 32), full(3, 32), full(3, 32, 32),
        ],
        out_specs=[
            pl.BlockSpec((_NB, 288), lambda i: (i, 0)),
            full(4, 32),
        ],
        out_shape=[
            jax.ShapeDtypeStruct((_N, 288), F32),
            jax.ShapeDtypeStruct((4, 32), F32),
        ],
    )(o1, st1, pw2d, plin)


def _nodeP3_body(o2_ref, st_ref, h0_ref, hs2_ref, hsc_ref):
    bks = _bn_apply(o2_ref, st_ref)
    b0 = bks[0] + h0_ref[:, 0:32]
    hs2_ref[...] = jnp.concatenate(
        [b0, jnp.zeros((b0.shape[0], 96), F32)], axis=1)
    for ci in range(2):
        hsc_ref[ci] = jnp.concatenate(
            [bks[k][:, ci * 16:(ci + 1) * 16] for k in range(1, 9)], axis=1)


def _nodeP3(o2, st2, tab):
    nb = _N // _NB
    full = lambda *s: pl.BlockSpec(s, lambda i: (0,) * len(s))
    return pl.pallas_call(
        _nodeP3_body,
        grid=(nb,),
        in_specs=[
            pl.BlockSpec((_NB, 288), lambda i: (i, 0)),
            full(4, 32),
            pl.BlockSpec((_NB, 128), lambda i: (i, 0)),
        ],
        out_specs=[
            pl.BlockSpec((_NB, 128), lambda i: (i, 0)),
            pl.BlockSpec((2, _NB, 128), lambda i: (0, i, 0)),
        ],
        out_shape=[
            jax.ShapeDtypeStruct((_N, 128), F32),
            jax.ShapeDtypeStruct((2, _N, 128), F32),
        ],
    )(o2, st2, tab)


def _nodeQ_body(o_ref, st_ref, pw_ref, plin_ref, q_ref, st2_ref):
    i = pl.program_id(0)
    aks = _bn_apply(o_ref, st_ref)
    a0 = aks[0]
    f = pw_ref[0:1, :] + pw_ref[1:2, :] * a0 + pw_ref[2:3, :] * a0 * a0
    q = jnp.dot(f * a0, plin_ref[0], preferred_element_type=F32)
    q_ref[...] = q
    part = jnp.concatenate([
        jnp.sum(q, axis=0, keepdims=True),
        jnp.sum(q * q, axis=0, keepdims=True)], axis=0)

    @pl.when(i == 0)
    def _():
        st2_ref[...] = part

    @pl.when(i != 0)
    def _():
        st2_ref[...] = st2_ref[...] + part


def _nodeQ(o1, st1, pw2d, plin):
    nb = _N // _NB
    full = lambda *s: pl.BlockSpec(s, lambda i: (0,) * len(s))
    return pl.pallas_call(
        _nodeQ_body,
        grid=(nb,),
        in_specs=[
            pl.BlockSpec((_NB, 288), lambda i: (i, 0)),
            full(4, 32), full(3, 32), full(3, 32, 32),
        ],
        out_specs=[
            pl.BlockSpec((_NB, 32), lambda i: (i, 0)),
            full(2, 32),
        ],
        out_shape=[
            jax.ShapeDtypeStruct((_N, 32), F32),
            jax.ShapeDtypeStruct((2, 32), F32),
        ],
    )(o1, st1, pw2d, plin)


def _nodeF_body(q_ref, st_ref, b_ref, node_ref, graph_ref):
    i = pl.program_id(0)
    mu = st_ref[0:1, :] * (1.0 / _N)
    var = st_ref[1:2, :] * (1.0 / _N) - mu * mu
    sd = jnp.sqrt(jnp.maximum(var, 0.0))
    node = (q_ref[...] - mu) / (sd + 1e-5)
    node_ref[...] = node
    bt = b_ref[0, 0, :]
    oht = (lax.broadcasted_iota(jnp.int32, (64, _NB), 0)
           == bt[None, :]).astype(F32)
    g = jnp.dot(oht, node, preferred_element_type=F32)

    @pl.when(i == 0)
    def _():
        graph_ref[...] = g

    @pl.when(i != 0)
    def _():
        graph_ref[...] = graph_ref[...] + g


def _nodeF(q, st3, batch3):
    nb = _N // _NB
    full = lambda *s: pl.BlockSpec(s, lambda i: (0,) * len(s))
    return pl.pallas_call(
        _nodeF_body,
        grid=(nb,),
        in_specs=[
            pl.BlockSpec((_NB, 32), lambda i: (i, 0)),
            full(2, 32),
            pl.BlockSpec((1, 1, _NB), lambda i: (i, 0, 0)),
        ],
        out_specs=[
            pl.BlockSpec((_NB, 32), lambda i: (i, 0)),
            full(64, 32),
        ],
        out_shape=[
            jax.ShapeDtypeStruct((_N, 32), F32),
            jax.ShapeDtypeStruct((64, 32), F32),
        ],
    )(q, st3, batch3)


def _perm_ws():
    return np.array([3 * c + l for l in range(3) for c in range(32)])


def _perm_wp():
    return np.array([2 * c + m for m in range(2) for c in range(32)])


_gather128 = _make_gather()
_scatter1 = _make_scatter(False)
_scatter2 = _make_scatter(True)
_scatter0 = _make_scatter0()


def kernel(x, pos, edge_index, batch, W_emb, b_emb, mW1_0, mb1_0, mW2_0,
           mb2_0, lin_0, pw_0, plin_0, mW1_1, mb1_1, mW2_1, mb2_1, lin_1,
           pw_1, plin_1):
    e0 = edge_index[0].astype(jnp.int32)
    e1 = edge_index[1].astype(jnp.int32)
    src_all = jnp.concatenate([e0, e1])
    dst_all = jnp.concatenate([e1, e0])

    pws = _perm_ws()
    pwp = _perm_wp()
    w1e_0, w1a_0, w1b_0 = mW1_0[0:8], mW1_0[8:40], mW1_0[40:72]
    w1e_1, w1a_1, w1b_1 = mW1_1[0:8], mW1_1[8:40], mW1_1[40:72]
    w2a_0 = mW2_0[:, pws]
    b2a_0 = mb2_0[pws][None, :]
    w2a_1 = mW2_1[:, :96][:, pws]
    b2a_1 = mb2_1[:96][pws][None, :]
    w2p_1 = mW2_1[:, 96:160][:, pwp]
    b2p_1 = mb2_1[96:160][pwp][None, :]

    tab = _embed(x, pos, W_emb, b_emb[None, :])
    g0, g1 = _gather128(tab, e0, e1)
    rec1, yx, ef = _edge1(g0, g1, w1e_0, w1a_0, w1b_0, mb1_0[None, :],
                          w2a_0, b2a_0)
    rec1 = rec1.reshape(2, _E2, 128)
    agg1 = _scatter1(rec1, dst_all)
    aggs1 = _scatter0(rec1, dst_all)
    o1, st1 = _nodeP1(agg1, aggs1, lin_0)
    o2, st2 = _nodeP2(o1, st1, pw_0, plin_0)
    hs2, h2sc = _nodeP3(o2, st2, tab)

    g20, g21 = _gather128(hs2, e0, e1)
    rec2 = _edge2(g20, g21, ef, yx, w1e_1, w1a_1, w1b_1,
                  mb1_1[None, :], w2a_1, b2a_1, w2p_1, b2p_1)
    rec2 = rec2.reshape(2, _E2, 128)
    agg2 = _scatter2(rec2, dst_all, src_all, h2sc[0], h2sc[1])
    aggs2 = _scatter0(rec2, dst_all)
    o1b, st1b = _nodeP1(agg2, aggs2, lin_1)
    q, st3 = _nodeQ(o1b, st1b, pw_1, plin_1)
    node, graph = _nodeF(q, st3,
                         batch.astype(jnp.int32).reshape(_N // _NB, 1, _NB))
    return (node, graph)
